# Initial kernel scaffold; baseline (speedup 1.0000x reference)
import jax
import jax.numpy as jnp
from jax.experimental import pallas as pl


def kernel(feats_s, edge_index_s, batch_s, labels_s, feats_t, edge_index_t, batch_t, W1_ps, b1_ps, W2_ps, b2_ps, W3_ps, b3_ps, W1_pt, b1_pt, W2_pt, b2_pt, W3_pt, b3_pt, W1_sh, b1_sh, W2_sh, b2_sh, W3_sh, b3_sh, Wc1, bc1, Wc2, bc2, Wd, bd):
    raise NotImplementedError("write your pallas kernel here")


# speedup vs baseline: 12.8149x; 12.8149x over previous
import functools

import jax
import jax.numpy as jnp
from jax import lax
from jax.experimental import pallas as pl
from jax.experimental.pallas import tpu as pltpu
from jax.experimental.pallas import tpu_sc as plsc

N = 4096
E = 131072
FDIM = 256
H1 = 128
H2 = 64
NG = 64
NC = 10
COEFF_DIFF = 0.1
COEFF_RECON = 0.1
COEFF_DOMAIN = 0.1
NORM = N * N / (2.0 * (N * N - E))

NCORE = 2
NSUB = 16
EPT = E // NSUB
DEG_CHUNK = 256
AGG_CHUNK = 128
PRED_CHUNK = 1024

def _mesh():
    return plsc.VectorSubcoreMesh(core_axis_name="c", subcore_axis_name="s",
                                  num_cores=NCORE, num_subcores=NSUB)


def _make_deg_kernel():
    def body(dst_all, out, acc, idx_i, ones_v, zero_v, sem):
        c = lax.axis_index("c")
        s = lax.axis_index("s")
        for k in range(DEG_CHUNK // 16):
            ones_v[pl.ds(16 * k, 16)] = jnp.ones((16,), jnp.float32)
            zero_v[pl.ds(16 * k, 16)] = jnp.zeros((16,), jnp.float32)
        zrow = s * (N // NSUB)
        pltpu.sync_copy(zero_v, acc.at[pl.ds(zrow, N // NSUB)])
        plsc.subcore_barrier()
        base = c * E + s * EPT

        def step(i, carry):
            pltpu.sync_copy(dst_all.at[pl.ds(base + i * DEG_CHUNK, DEG_CHUNK)],
                            idx_i)
            pltpu.async_copy(ones_v, acc.at[idx_i], sem, add=True).wait()
            return carry

        lax.fori_loop(0, EPT // DEG_CHUNK, step, 0)
        plsc.subcore_barrier()
        pltpu.sync_copy(acc.at[pl.ds(zrow, N // NSUB)],
                        out.at[c, pl.ds(zrow, N // NSUB)])

    return pl.kernel(
        body,
        out_type=jax.ShapeDtypeStruct((NCORE, N), jnp.float32),
        mesh=_mesh(),
        scratch_types=[
            pltpu.VMEM_SHARED((N,), jnp.float32),
            pltpu.VMEM((DEG_CHUNK,), jnp.int32),
            pltpu.VMEM((DEG_CHUNK,), jnp.float32),
            pltpu.VMEM((N // NSUB,), jnp.float32),
            pltpu.SemaphoreType.DMA,
        ],
    )


def _make_agg_kernel():
    width = H1
    rows_per_tile = N // NSUB

    def body(hp_a, hp_b, src_all, dst_all, zrows, out_a, out_b,
             acc_a, acc_b, idx_s, idx_d, buf_a, buf_b, zb, sem):
        c = lax.axis_index("c")
        s = lax.axis_index("s")
        pltpu.sync_copy(zrows, zb)
        for k in range(rows_per_tile // 16):
            pltpu.sync_copy(zb, acc_a.at[pl.ds(s * rows_per_tile + 16 * k, 16)])
            pltpu.sync_copy(zb, acc_b.at[pl.ds(s * rows_per_tile + 16 * k, 16)])
        plsc.subcore_barrier()
        base = c * E + s * EPT

        def step(i, carry):
            pltpu.sync_copy(src_all.at[pl.ds(base + i * AGG_CHUNK, AGG_CHUNK)],
                            idx_s)
            pltpu.sync_copy(dst_all.at[pl.ds(base + i * AGG_CHUNK, AGG_CHUNK)],
                            idx_d)
            pltpu.async_copy(hp_a.at[idx_s], buf_a, sem).wait()
            pltpu.async_copy(buf_a, acc_a.at[idx_d], sem, add=True).wait()
            pltpu.async_copy(hp_b.at[idx_s], buf_b, sem).wait()
            pltpu.async_copy(buf_b, acc_b.at[idx_d], sem, add=True).wait()
            return carry

        lax.fori_loop(0, EPT // AGG_CHUNK, step, 0)
        plsc.subcore_barrier()
        pltpu.sync_copy(acc_a.at[pl.ds(s * rows_per_tile, rows_per_tile)],
                        out_a.at[c, pl.ds(s * rows_per_tile, rows_per_tile)])
        pltpu.sync_copy(acc_b.at[pl.ds(s * rows_per_tile, rows_per_tile)],
                        out_b.at[c, pl.ds(s * rows_per_tile, rows_per_tile)])

    return pl.kernel(
        body,
        out_type=(jax.ShapeDtypeStruct((NCORE, N, width), jnp.float32),
                  jax.ShapeDtypeStruct((NCORE, N, width), jnp.float32)),
        mesh=_mesh(),
        scratch_types=[
            pltpu.VMEM_SHARED((N, width), jnp.float32),
            pltpu.VMEM_SHARED((N, width), jnp.float32),
            pltpu.VMEM((AGG_CHUNK,), jnp.int32),
            pltpu.VMEM((AGG_CHUNK,), jnp.int32),
            pltpu.VMEM((AGG_CHUNK, width), jnp.float32),
            pltpu.VMEM((AGG_CHUNK, width), jnp.float32),
            pltpu.VMEM((16, width), jnp.float32),
            pltpu.SemaphoreType.DMA,
        ],
    )


def _make_pred_kernel():
    ept = (2 * E) // NSUB

    def body(recd_flat, idx_all, out, idx_v, val_v, sem):
        c = lax.axis_index("c")
        s = lax.axis_index("s")
        base = c * (2 * E) + s * ept

        def step(i, carry):
            off = base + i * PRED_CHUNK
            pltpu.sync_copy(idx_all.at[pl.ds(off, PRED_CHUNK)], idx_v)
            pltpu.async_copy(recd_flat.at[idx_v], val_v, sem).wait()
            pltpu.sync_copy(val_v, out.at[pl.ds(off, PRED_CHUNK)])
            return carry

        lax.fori_loop(0, ept // PRED_CHUNK, step, 0)

    return pl.kernel(
        body,
        out_type=jax.ShapeDtypeStruct((4 * E,), jnp.float32),
        mesh=_mesh(),
        scratch_types=[
            pltpu.VMEM((PRED_CHUNK,), jnp.int32),
            pltpu.VMEM((PRED_CHUNK,), jnp.float32),
            pltpu.SemaphoreType.DMA,
        ],
    )


_SC_CACHE = {}


def _deg_kernel(dst_all):
    if "deg" not in _SC_CACHE:
        _SC_CACHE["deg"] = _make_deg_kernel()
    return _SC_CACHE["deg"](dst_all)


def _agg_kernel(hp_a, hp_b, src_all, dst_all, zrows):
    if "agg" not in _SC_CACHE:
        _SC_CACHE["agg"] = _make_agg_kernel()
    return _SC_CACHE["agg"](hp_a, hp_b, src_all, dst_all, zrows)


def _pred_kernel(recd_flat, idx_all):
    if "pred" not in _SC_CACHE:
        _SC_CACHE["pred"] = _make_pred_kernel()
    return _SC_CACHE["pred"](recd_flat, idx_all)


_RB = 512


def _spec_h(gph_i):
    return pl.BlockSpec((1, _RB, H1), lambda gph, i: (gph, i, 0))


def _m1_body(x_ref, w_ref, b_ref, dinv_ref, oa_ref, ob_ref):
    h = jnp.dot(x_ref[0], w_ref[0], preferred_element_type=jnp.float32)
    o = dinv_ref[0] * (h + b_ref[0])
    oa_ref[0] = o[:, :H1]
    ob_ref[0] = o[:, H1:]


def _m1(x, w, b, dinv):
    g = N // _RB
    return pl.pallas_call(
        _m1_body,
        grid=(2, g),
        in_specs=[
            pl.BlockSpec((1, _RB, FDIM), lambda gph, i: (gph, i, 0)),
            pl.BlockSpec((1, FDIM, 2 * H1), lambda gph, i: (gph, 0, 0)),
            pl.BlockSpec((1, 1, 2 * H1), lambda gph, i: (gph, 0, 0)),
            pl.BlockSpec((1, _RB, 1), lambda gph, i: (gph, i, 0)),
        ],
        out_specs=[_spec_h(None), _spec_h(None)],
        out_shape=[jax.ShapeDtypeStruct((2, N, H1), jnp.float32),
                   jax.ShapeDtypeStruct((2, N, H1), jnp.float32)],
    )(x, w, b, dinv)


def _m2_body(ra_ref, rb_ref, ha_ref, hb_ref, dinv_ref, w_ref, b_ref,
             oa_ref, ob_ref):
    dinv = dinv_ref[0]
    h1a = jnp.maximum(dinv * (ra_ref[0] + ha_ref[0]), 0.0)
    h1b = jnp.maximum(dinv * (rb_ref[0] + hb_ref[0]), 0.0)
    w = w_ref[0]
    h = (jnp.dot(h1a, w[:H1], preferred_element_type=jnp.float32)
         + jnp.dot(h1b, w[H1:], preferred_element_type=jnp.float32))
    o = dinv * (h + b_ref[0])
    oa_ref[0] = o[:, :H1]
    ob_ref[0] = o[:, H1:]


def _m2(ra, rb, ha, hb, dinv, w, b):
    g = N // _RB
    return pl.pallas_call(
        _m2_body,
        grid=(2, g),
        in_specs=[
            _spec_h(None), _spec_h(None), _spec_h(None), _spec_h(None),
            pl.BlockSpec((1, _RB, 1), lambda gph, i: (gph, i, 0)),
            pl.BlockSpec((1, 2 * H1, 2 * H1), lambda gph, i: (gph, 0, 0)),
            pl.BlockSpec((1, 1, 2 * H1), lambda gph, i: (gph, 0, 0)),
        ],
        out_specs=[_spec_h(None), _spec_h(None)],
        out_shape=[jax.ShapeDtypeStruct((2, N, H1), jnp.float32),
                   jax.ShapeDtypeStruct((2, N, H1), jnp.float32)],
    )(ra, rb, ha, hb, dinv, w, b)


def _m3_body(ra_ref, rb_ref, ha_ref, hb_ref, dinv_ref, eps_ref,
             mulv_ref, zcat_ref):
    dinv = dinv_ref[0]
    ma = dinv * (ra_ref[0] + ha_ref[0])
    mb = dinv * (rb_ref[0] + hb_ref[0])
    mulv_ref[0] = jnp.concatenate([ma, mb], axis=1)
    eps = eps_ref[...]
    zp = ma[:, :H2] + eps * jnp.exp(ma[:, H2:])
    zh = mb[:, :H2] + eps * jnp.exp(mb[:, H2:])
    zcat_ref[0] = jnp.concatenate([zp, zh], axis=1)


def _m3(ra, rb, ha, hb, dinv, eps):
    g = N // _RB
    return pl.pallas_call(
        _m3_body,
        grid=(2, g),
        in_specs=[
            _spec_h(None), _spec_h(None), _spec_h(None), _spec_h(None),
            pl.BlockSpec((1, _RB, 1), lambda gph, i: (gph, i, 0)),
            pl.BlockSpec((_RB, H2), lambda gph, i: (i, 0)),
        ],
        out_specs=[
            pl.BlockSpec((1, _RB, 2 * H1), lambda gph, i: (gph, i, 0)),
            pl.BlockSpec((1, _RB, H1), lambda gph, i: (gph, i, 0)),
        ],
        out_shape=[
            jax.ShapeDtypeStruct((2, N, 2 * H1), jnp.float32),
            jax.ShapeDtypeStruct((2, N, H1), jnp.float32),
        ],
    )(ra, rb, ha, hb, dinv, eps)


def _m4_body(a_ref, b_ref, o_ref):
    o_ref[0] = lax.dot_general(a_ref[0], b_ref[0],
                               (((1,), (1,)), ((), ())),
                               preferred_element_type=jnp.float32)


def _m4(zcat):
    g = N // _RB
    return pl.pallas_call(
        _m4_body,
        grid=(2, g, g),
        in_specs=[
            pl.BlockSpec((1, _RB, H1), lambda gph, i, j: (gph, i, 0)),
            pl.BlockSpec((1, _RB, H1), lambda gph, i, j: (gph, j, 0)),
        ],
        out_specs=pl.BlockSpec((1, _RB, _RB), lambda gph, i, j: (gph, i, j)),
        out_shape=jax.ShapeDtypeStruct((2, N, N), jnp.float32),
    )(zcat, zcat)


def _m5_body(mulv_ref, preds_ref, bs_ref, bt_ref, lab_ref,
             wc1_ref, bc1_ref, wc2_ref, bc2_ref, wd_ref, bd_ref, o_ref):
    mul_s = mulv_ref[0]
    mul_t = mulv_ref[1]

    preds = preds_ref[...]
    softplus = jnp.log(1.0 + jnp.exp(-jnp.abs(preds)))
    base = jnp.maximum(preds, 0.0) + softplus
    rows = E // H1
    s_pos = jnp.sum(base[:2 * rows])
    s_neg = jnp.sum(base[2 * rows:])
    cost_s = NORM * (s_pos - jnp.sum(preds[:rows])) / (2 * E)
    cost_t = NORM * (s_neg - jnp.sum(preds[2 * rows:3 * rows])) / (2 * E)

    def kld(m, denom):
        mu_p, lv_p = m[:, :H2], m[:, H2:2 * H2]
        mu_h, lv_h = m[:, 2 * H2:3 * H2], m[:, 3 * H2:]
        t = (1.0 + 2.0 * lv_p - mu_p * mu_p - jnp.exp(lv_p) ** 2
             + 1.0 + 2.0 * lv_h - mu_h * mu_h - jnp.exp(lv_h) ** 2)
        return -0.5 / denom * jnp.sum(t) / N

    recon = cost_s + kld(mul_s, N) + cost_t + kld(mul_t, 2 * N)

    sh_s1 = mul_s[:, 2 * H2:3 * H2]
    sh_t1 = mul_t[:, 2 * H2:3 * H2]
    iota_g = lax.broadcasted_iota(jnp.int32, (N, NG), 1)
    oh_s = (bs_ref[...] == iota_g).astype(jnp.float32)
    oh_t = (bt_ref[...] == iota_g).astype(jnp.float32)
    ones_c = jnp.ones((N, 1), jnp.float32)
    cnt_s = jnp.clip(lax.dot_general(oh_s, ones_c, (((0,), (0,)), ((), ())),
                                     preferred_element_type=jnp.float32),
                     1.0, None)
    cnt_t = jnp.clip(lax.dot_general(oh_t, ones_c, (((0,), (0,)), ((), ())),
                                     preferred_element_type=jnp.float32),
                     1.0, None)
    pool_s = lax.dot_general(oh_s, sh_s1, (((0,), (0,)), ((), ())),
                             preferred_element_type=jnp.float32) / cnt_s
    pool_t = lax.dot_general(oh_t, sh_t1, (((0,), (0,)), ((), ())),
                             preferred_element_type=jnp.float32) / cnt_t

    lab1h = (lab_ref[...] == lax.broadcasted_iota(jnp.int32, (NG, NC), 1)
             ).astype(jnp.float32)
    hc = jnp.maximum(
        jnp.dot(pool_s, wc1_ref[...], preferred_element_type=jnp.float32)
        + bc1_ref[...], 0.0)
    logits = (jnp.dot(hc, wc2_ref[...], preferred_element_type=jnp.float32)
              + bc2_ref[...])
    p = 1.0 / (1.0 + jnp.exp(-logits))
    p = jnp.clip(p, 1e-07, 1.0 - 1e-07)
    clf = -jnp.mean(lab1h * jnp.log(p) + (1.0 - lab1h) * jnp.log(1.0 - p))

    def dloss(a, b):
        an = jnp.sqrt(jnp.sum(a * a, axis=1, keepdims=True))
        bn = jnp.sqrt(jnp.sum(b * b, axis=1, keepdims=True))
        a2 = a / (an + 1e-06)
        b2 = b / (bn + 1e-06)
        cmat = lax.dot_general(a2, b2, (((0,), (0,)), ((), ())),
                               preferred_element_type=jnp.float32)
        return jnp.sum(cmat * cmat) / (H2 * H2)

    diff = dloss(mul_s[:, :H2], sh_s1) + dloss(mul_t[:, :H2], sh_t1)

    dp_s = 1.0 / (1.0 + jnp.exp(-(jnp.dot(pool_s, wd_ref[...],
                                          preferred_element_type=jnp.float32)
                                  + bd_ref[...])))
    dp_t = 1.0 / (1.0 + jnp.exp(-(jnp.dot(pool_t, wd_ref[...],
                                          preferred_element_type=jnp.float32)
                                  + bd_ref[...])))
    dp_s = jnp.clip(dp_s, 1e-07, 1.0 - 1e-07)
    dp_t = jnp.clip(dp_t, 1e-07, 1.0 - 1e-07)
    domain = (-jnp.mean(jnp.log(1.0 - dp_s))) + (-jnp.mean(jnp.log(dp_t)))

    total = (clf + COEFF_DIFF * diff + COEFF_RECON * recon
             + COEFF_DOMAIN * domain)
    o_ref[...] = jnp.reshape(total, (1, 1))


def _m5(mulv, preds, bs, bt, lab, wc1, bc1, wc2, bc2, wd, bd):
    return pl.pallas_call(
        _m5_body,
        out_shape=jax.ShapeDtypeStruct((1, 1), jnp.float32),
    )(mulv, preds, bs, bt, lab, wc1, bc1, wc2, bc2, wd, bd)


def kernel(feats_s, edge_index_s, batch_s, labels_s, feats_t, edge_index_t,
           batch_t, W1_ps, b1_ps, W2_ps, b2_ps, W3_ps, b3_ps,
           W1_pt, b1_pt, W2_pt, b2_pt, W3_pt, b3_pt,
           W1_sh, b1_sh, W2_sh, b2_sh, W3_sh, b3_sh,
           Wc1, bc1, Wc2, bc2, Wd, bd):
    f32 = jnp.float32
    eps = jax.random.normal(jax.random.key(42), (N, H2), f32)
    neg_s = jax.random.randint(jax.random.key(7), (2, E), 0, N)
    neg_t = jax.random.randint(jax.random.key(8), (2, E), 0, N)

    src_s = edge_index_s[0].astype(jnp.int32)
    dst_s = edge_index_s[1].astype(jnp.int32)
    src_t = edge_index_t[0].astype(jnp.int32)
    dst_t = edge_index_t[1].astype(jnp.int32)

    dst_all = jnp.concatenate([dst_s, dst_t])
    src_all = jnp.concatenate([src_s, src_t + N])

    deg = _deg_kernel(dst_all)
    dinv = lax.rsqrt(jnp.clip(deg + 1.0, 1.0, None)).reshape(2, N, 1)

    W1c = jnp.stack([jnp.concatenate([W1_ps, W1_sh], axis=1),
                     jnp.concatenate([W1_pt, W1_sh], axis=1)])
    b1c = jnp.stack([jnp.concatenate([b1_ps, b1_sh]).reshape(1, 2 * H1),
                     jnp.concatenate([b1_pt, b1_sh]).reshape(1, 2 * H1)])
    Z = jnp.zeros((H1, H2), f32)

    def blk(w2a, w3a):
        return jnp.concatenate([w2a, w3a, Z, Z], axis=1)

    def blk2(w2b, w3b):
        return jnp.concatenate([Z, Z, w2b, w3b], axis=1)

    Wblk = jnp.stack([
        jnp.concatenate([blk(W2_ps, W3_ps), blk2(W2_sh, W3_sh)], axis=0),
        jnp.concatenate([blk(W2_pt, W3_pt), blk2(W2_sh, W3_sh)], axis=0)])
    bblk = jnp.stack([
        jnp.concatenate([b2_ps, b3_ps, b2_sh, b3_sh]).reshape(1, 2 * H1),
        jnp.concatenate([b2_pt, b3_pt, b2_sh, b3_sh]).reshape(1, 2 * H1)])

    xs = jnp.stack([feats_s, feats_t])
    zrows = jnp.zeros((16, H1), f32)

    hp1a, hp1b = _m1(xs, W1c, b1c, dinv)
    raw1a, raw1b = _agg_kernel(hp1a.reshape(2 * N, H1),
                               hp1b.reshape(2 * N, H1),
                               src_all, dst_all, zrows)
    hp2a, hp2b = _m2(raw1a, raw1b, hp1a, hp1b, dinv, Wblk, bblk)
    raw2a, raw2b = _agg_kernel(hp2a.reshape(2 * N, H1),
                               hp2b.reshape(2 * N, H1),
                               src_all, dst_all, zrows)
    mulv, zcat = _m3(raw2a, raw2b, hp2a, hp2b, dinv, eps)

    recd = _m4(zcat)

    idx_all = jnp.concatenate([
        src_s * N + dst_s,
        neg_s[0] * N + neg_s[1],
        N * N + src_t * N + dst_t,
        N * N + neg_t[0] * N + neg_t[1]]).astype(jnp.int32)
    preds = _pred_kernel(recd.reshape(2 * N * N), idx_all)

    out = _m5(mulv, preds.reshape(E // 32, H1),
              batch_s.astype(jnp.int32).reshape(N, 1),
              batch_t.astype(jnp.int32).reshape(N, 1),
              labels_s.astype(jnp.int32).reshape(NG, 1),
              Wc1, bc1.reshape(1, 16), Wc2, bc2.reshape(1, NC),
              Wd, bd.reshape(1, 1))
    return out.reshape(())


# speedup vs baseline: 17.2370x; 1.3451x over previous
import functools

import jax
import jax.numpy as jnp
from jax import lax
from jax.experimental import pallas as pl
from jax.experimental.pallas import tpu as pltpu
from jax.experimental.pallas import tpu_sc as plsc

N = 4096
E = 131072
FDIM = 256
H1 = 128
H2 = 64
NG = 64
NC = 10
COEFF_DIFF = 0.1
COEFF_RECON = 0.1
COEFF_DOMAIN = 0.1
NORM = N * N / (2.0 * (N * N - E))

NCORE = 2
NSUB = 16
EPT = E // NSUB
DEG_CHUNK = 256
AGG_CHUNK = 128
PRED_CHUNK = 1024

def _mesh():
    return plsc.VectorSubcoreMesh(core_axis_name="c", subcore_axis_name="s",
                                  num_cores=NCORE, num_subcores=NSUB)


def _make_deg_kernel():
    def body(dst_all, out, acc, idx_i, ones_v, zero_v, sem):
        c = lax.axis_index("c")
        s = lax.axis_index("s")
        for k in range(DEG_CHUNK // 16):
            ones_v[pl.ds(16 * k, 16)] = jnp.ones((16,), jnp.float32)
            zero_v[pl.ds(16 * k, 16)] = jnp.zeros((16,), jnp.float32)
        zrow = s * (N // NSUB)
        pltpu.sync_copy(zero_v, acc.at[pl.ds(zrow, N // NSUB)])
        plsc.subcore_barrier()
        base = c * E + s * EPT

        def step(i, carry):
            pltpu.sync_copy(dst_all.at[pl.ds(base + i * DEG_CHUNK, DEG_CHUNK)],
                            idx_i)
            pltpu.async_copy(ones_v, acc.at[idx_i], sem, add=True).wait()
            return carry

        lax.fori_loop(0, EPT // DEG_CHUNK, step, 0)
        plsc.subcore_barrier()
        pltpu.sync_copy(acc.at[pl.ds(zrow, N // NSUB)],
                        out.at[c, pl.ds(zrow, N // NSUB)])

    return pl.kernel(
        body,
        out_type=jax.ShapeDtypeStruct((NCORE, N), jnp.float32),
        mesh=_mesh(),
        scratch_types=[
            pltpu.VMEM_SHARED((N,), jnp.float32),
            pltpu.VMEM((DEG_CHUNK,), jnp.int32),
            pltpu.VMEM((DEG_CHUNK,), jnp.float32),
            pltpu.VMEM((N // NSUB,), jnp.float32),
            pltpu.SemaphoreType.DMA,
        ],
    )


AGG_NB = 4


def _make_agg_kernel():
    width = H1
    rows_per_tile = N // NSUB
    chunks = EPT // AGG_CHUNK

    def body(hp_a, hp_b, src_all, dst_all, zrows, out_a, out_b,
             acc, i0, i1, i2, i3, j0, j1, j2, j3, b0, b1, b2, b3, zb,
             semi, semg, sema):
        c = lax.axis_index("c")
        s = lax.axis_index("s")
        bufs = (b0, b1, b2, b3)
        idxs = (i0, i1, i2, i3)
        idxd = (j0, j1, j2, j3)
        pltpu.sync_copy(zrows, zb)
        base = c * E + s * EPT
        for phase in range(2):
            hp_h = (hp_a, hp_b)[phase]
            out_h = (out_a, out_b)[phase]
            for k in range(rows_per_tile // 16):
                pltpu.sync_copy(zb, acc.at[pl.ds(s * rows_per_tile + 16 * k,
                                                 16)])
            plsc.subcore_barrier()

            def step(i, carry):
                ls = []
                for b in range(AGG_NB):
                    off = base + (i * AGG_NB + b) * AGG_CHUNK
                    ls.append(pltpu.async_copy(
                        src_all.at[pl.ds(off, AGG_CHUNK)], idxs[b], semi))
                    ls.append(pltpu.async_copy(
                        dst_all.at[pl.ds(off, AGG_CHUNK)], idxd[b], semi))
                gs = []
                for b in range(AGG_NB):
                    ls[2 * b].wait()
                    ls[2 * b + 1].wait()
                    gs.append(pltpu.async_copy(hp_h.at[idxs[b]], bufs[b],
                                               semg))
                ads = []
                for b in range(AGG_NB):
                    gs[b].wait()
                    ads.append(pltpu.async_copy(bufs[b], acc.at[idxd[b]],
                                                sema, add=True))
                for b in range(AGG_NB):
                    ads[b].wait()
                return carry

            lax.fori_loop(0, chunks // AGG_NB, step, 0)
            plsc.subcore_barrier()
            pltpu.sync_copy(acc.at[pl.ds(s * rows_per_tile, rows_per_tile)],
                            out_h.at[c, pl.ds(s * rows_per_tile,
                                              rows_per_tile)])
            plsc.subcore_barrier()

    return pl.kernel(
        body,
        out_type=(jax.ShapeDtypeStruct((NCORE, N, width), jnp.float32),
                  jax.ShapeDtypeStruct((NCORE, N, width), jnp.float32)),
        mesh=_mesh(),
        scratch_types=(
            [pltpu.VMEM_SHARED((N, width), jnp.float32)]
            + [pltpu.VMEM((AGG_CHUNK,), jnp.int32) for _ in range(8)]
            + [pltpu.VMEM((AGG_CHUNK, width), jnp.float32) for _ in range(4)]
            + [pltpu.VMEM((16, width), jnp.float32),
               pltpu.SemaphoreType.DMA,
               pltpu.SemaphoreType.DMA,
               pltpu.SemaphoreType.DMA]
        ),
    )


def _make_pred_kernel():
    ept = (2 * E) // NSUB

    def body(recd_flat, idx_all, out, idx_v, val_v, sem):
        c = lax.axis_index("c")
        s = lax.axis_index("s")
        base = c * (2 * E) + s * ept

        def step(i, carry):
            off = base + i * PRED_CHUNK
            pltpu.sync_copy(idx_all.at[pl.ds(off, PRED_CHUNK)], idx_v)
            pltpu.async_copy(recd_flat.at[idx_v], val_v, sem).wait()
            pltpu.sync_copy(val_v, out.at[pl.ds(off, PRED_CHUNK)])
            return carry

        lax.fori_loop(0, ept // PRED_CHUNK, step, 0)

    return pl.kernel(
        body,
        out_type=jax.ShapeDtypeStruct((4 * E,), jnp.float32),
        mesh=_mesh(),
        scratch_types=[
            pltpu.VMEM((PRED_CHUNK,), jnp.int32),
            pltpu.VMEM((PRED_CHUNK,), jnp.float32),
            pltpu.SemaphoreType.DMA,
        ],
    )


_SC_CACHE = {}


def _deg_kernel(dst_all):
    if "deg" not in _SC_CACHE:
        _SC_CACHE["deg"] = _make_deg_kernel()
    return _SC_CACHE["deg"](dst_all)


def _agg_kernel(hp_a, hp_b, src2d, dst2d, zrows):
    if "agg" not in _SC_CACHE:
        _SC_CACHE["agg"] = _make_agg_kernel()
    return _SC_CACHE["agg"](hp_a, hp_b, src2d, dst2d, zrows)


def _pred_kernel(recd_flat, idx_all):
    if "pred" not in _SC_CACHE:
        _SC_CACHE["pred"] = _make_pred_kernel()
    return _SC_CACHE["pred"](recd_flat, idx_all)


_RB = 512


def _spec_h(gph_i):
    return pl.BlockSpec((1, _RB, H1), lambda gph, i: (gph, i, 0))


def _m1_body(x_ref, w_ref, b_ref, dinv_ref, oa_ref, ob_ref):
    h = jnp.dot(x_ref[0], w_ref[0], preferred_element_type=jnp.float32)
    o = dinv_ref[0] * (h + b_ref[0])
    oa_ref[0] = o[:, :H1]
    ob_ref[0] = o[:, H1:]


def _m1(x, w, b, dinv):
    g = N // _RB
    return pl.pallas_call(
        _m1_body,
        grid=(2, g),
        in_specs=[
            pl.BlockSpec((1, _RB, FDIM), lambda gph, i: (gph, i, 0)),
            pl.BlockSpec((1, FDIM, 2 * H1), lambda gph, i: (gph, 0, 0)),
            pl.BlockSpec((1, 1, 2 * H1), lambda gph, i: (gph, 0, 0)),
            pl.BlockSpec((1, _RB, 1), lambda gph, i: (gph, i, 0)),
        ],
        out_specs=[_spec_h(None), _spec_h(None)],
        out_shape=[jax.ShapeDtypeStruct((2, N, H1), jnp.float32),
                   jax.ShapeDtypeStruct((2, N, H1), jnp.float32)],
    )(x, w, b, dinv)


def _m2_body(ra_ref, rb_ref, ha_ref, hb_ref, dinv_ref, w_ref, b_ref,
             oa_ref, ob_ref):
    dinv = dinv_ref[0]
    h1a = jnp.maximum(dinv * (ra_ref[0] + ha_ref[0]), 0.0)
    h1b = jnp.maximum(dinv * (rb_ref[0] + hb_ref[0]), 0.0)
    w = w_ref[0]
    h = (jnp.dot(h1a, w[:H1], preferred_element_type=jnp.float32)
         + jnp.dot(h1b, w[H1:], preferred_element_type=jnp.float32))
    o = dinv * (h + b_ref[0])
    oa_ref[0] = o[:, :H1]
    ob_ref[0] = o[:, H1:]


def _m2(ra, rb, ha, hb, dinv, w, b):
    g = N // _RB
    return pl.pallas_call(
        _m2_body,
        grid=(2, g),
        in_specs=[
            _spec_h(None), _spec_h(None), _spec_h(None), _spec_h(None),
            pl.BlockSpec((1, _RB, 1), lambda gph, i: (gph, i, 0)),
            pl.BlockSpec((1, 2 * H1, 2 * H1), lambda gph, i: (gph, 0, 0)),
            pl.BlockSpec((1, 1, 2 * H1), lambda gph, i: (gph, 0, 0)),
        ],
        out_specs=[_spec_h(None), _spec_h(None)],
        out_shape=[jax.ShapeDtypeStruct((2, N, H1), jnp.float32),
                   jax.ShapeDtypeStruct((2, N, H1), jnp.float32)],
    )(ra, rb, ha, hb, dinv, w, b)


def _m3_body(ra_ref, rb_ref, ha_ref, hb_ref, dinv_ref, eps_ref,
             mulv_ref, zcat_ref):
    dinv = dinv_ref[0]
    ma = dinv * (ra_ref[0] + ha_ref[0])
    mb = dinv * (rb_ref[0] + hb_ref[0])
    mulv_ref[0] = jnp.concatenate([ma, mb], axis=1)
    eps = eps_ref[...]
    zp = ma[:, :H2] + eps * jnp.exp(ma[:, H2:])
    zh = mb[:, :H2] + eps * jnp.exp(mb[:, H2:])
    zcat_ref[0] = jnp.concatenate([zp, zh], axis=1)


def _m3(ra, rb, ha, hb, dinv, eps):
    g = N // _RB
    return pl.pallas_call(
        _m3_body,
        grid=(2, g),
        in_specs=[
            _spec_h(None), _spec_h(None), _spec_h(None), _spec_h(None),
            pl.BlockSpec((1, _RB, 1), lambda gph, i: (gph, i, 0)),
            pl.BlockSpec((_RB, H2), lambda gph, i: (i, 0)),
        ],
        out_specs=[
            pl.BlockSpec((1, _RB, 2 * H1), lambda gph, i: (gph, i, 0)),
            pl.BlockSpec((1, _RB, H1), lambda gph, i: (gph, i, 0)),
        ],
        out_shape=[
            jax.ShapeDtypeStruct((2, N, 2 * H1), jnp.float32),
            jax.ShapeDtypeStruct((2, N, H1), jnp.float32),
        ],
    )(ra, rb, ha, hb, dinv, eps)


def _m4_body(a_ref, b_ref, o_ref):
    o_ref[0] = lax.dot_general(a_ref[0], b_ref[0],
                               (((1,), (1,)), ((), ())),
                               preferred_element_type=jnp.float32)


def _m4(zcat):
    g = N // _RB
    return pl.pallas_call(
        _m4_body,
        grid=(2, g, g),
        in_specs=[
            pl.BlockSpec((1, _RB, H1), lambda gph, i, j: (gph, i, 0)),
            pl.BlockSpec((1, _RB, H1), lambda gph, i, j: (gph, j, 0)),
        ],
        out_specs=pl.BlockSpec((1, _RB, _RB), lambda gph, i, j: (gph, i, j)),
        out_shape=jax.ShapeDtypeStruct((2, N, N), jnp.float32),
    )(zcat, zcat)


def _m5_body(mulv_ref, preds_ref, bs_ref, bt_ref, lab_ref,
             wc1_ref, bc1_ref, wc2_ref, bc2_ref, wd_ref, bd_ref, o_ref):
    mul_s = mulv_ref[0]
    mul_t = mulv_ref[1]

    preds = preds_ref[...]
    softplus = jnp.log(1.0 + jnp.exp(-jnp.abs(preds)))
    base = jnp.maximum(preds, 0.0) + softplus
    rows = E // H1
    s_pos = jnp.sum(base[:2 * rows])
    s_neg = jnp.sum(base[2 * rows:])
    cost_s = NORM * (s_pos - jnp.sum(preds[:rows])) / (2 * E)
    cost_t = NORM * (s_neg - jnp.sum(preds[2 * rows:3 * rows])) / (2 * E)

    def kld(m, denom):
        mu_p, lv_p = m[:, :H2], m[:, H2:2 * H2]
        mu_h, lv_h = m[:, 2 * H2:3 * H2], m[:, 3 * H2:]
        t = (1.0 + 2.0 * lv_p - mu_p * mu_p - jnp.exp(lv_p) ** 2
             + 1.0 + 2.0 * lv_h - mu_h * mu_h - jnp.exp(lv_h) ** 2)
        return -0.5 / denom * jnp.sum(t) / N

    recon = cost_s + kld(mul_s, N) + cost_t + kld(mul_t, 2 * N)

    sh_s1 = mul_s[:, 2 * H2:3 * H2]
    sh_t1 = mul_t[:, 2 * H2:3 * H2]
    iota_g = lax.broadcasted_iota(jnp.int32, (N, NG), 1)
    oh_s = (bs_ref[...] == iota_g).astype(jnp.float32)
    oh_t = (bt_ref[...] == iota_g).astype(jnp.float32)
    ones_c = jnp.ones((N, 1), jnp.float32)
    cnt_s = jnp.clip(lax.dot_general(oh_s, ones_c, (((0,), (0,)), ((), ())),
                                     preferred_element_type=jnp.float32),
                     1.0, None)
    cnt_t = jnp.clip(lax.dot_general(oh_t, ones_c, (((0,), (0,)), ((), ())),
                                     preferred_element_type=jnp.float32),
                     1.0, None)
    pool_s = lax.dot_general(oh_s, sh_s1, (((0,), (0,)), ((), ())),
                             preferred_element_type=jnp.float32) / cnt_s
    pool_t = lax.dot_general(oh_t, sh_t1, (((0,), (0,)), ((), ())),
                             preferred_element_type=jnp.float32) / cnt_t

    lab1h = (lab_ref[...] == lax.broadcasted_iota(jnp.int32, (NG, NC), 1)
             ).astype(jnp.float32)
    hc = jnp.maximum(
        jnp.dot(pool_s, wc1_ref[...], preferred_element_type=jnp.float32)
        + bc1_ref[...], 0.0)
    logits = (jnp.dot(hc, wc2_ref[...], preferred_element_type=jnp.float32)
              + bc2_ref[...])
    p = 1.0 / (1.0 + jnp.exp(-logits))
    p = jnp.clip(p, 1e-07, 1.0 - 1e-07)
    clf = -jnp.mean(lab1h * jnp.log(p) + (1.0 - lab1h) * jnp.log(1.0 - p))

    def dloss(a, b):
        an = jnp.sqrt(jnp.sum(a * a, axis=1, keepdims=True))
        bn = jnp.sqrt(jnp.sum(b * b, axis=1, keepdims=True))
        a2 = a / (an + 1e-06)
        b2 = b / (bn + 1e-06)
        cmat = lax.dot_general(a2, b2, (((0,), (0,)), ((), ())),
                               preferred_element_type=jnp.float32)
        return jnp.sum(cmat * cmat) / (H2 * H2)

    diff = dloss(mul_s[:, :H2], sh_s1) + dloss(mul_t[:, :H2], sh_t1)

    dp_s = 1.0 / (1.0 + jnp.exp(-(jnp.dot(pool_s, wd_ref[...],
                                          preferred_element_type=jnp.float32)
                                  + bd_ref[...])))
    dp_t = 1.0 / (1.0 + jnp.exp(-(jnp.dot(pool_t, wd_ref[...],
                                          preferred_element_type=jnp.float32)
                                  + bd_ref[...])))
    dp_s = jnp.clip(dp_s, 1e-07, 1.0 - 1e-07)
    dp_t = jnp.clip(dp_t, 1e-07, 1.0 - 1e-07)
    domain = (-jnp.mean(jnp.log(1.0 - dp_s))) + (-jnp.mean(jnp.log(dp_t)))

    total = (clf + COEFF_DIFF * diff + COEFF_RECON * recon
             + COEFF_DOMAIN * domain)
    o_ref[...] = jnp.reshape(total, (1, 1))


def _m5(mulv, preds, bs, bt, lab, wc1, bc1, wc2, bc2, wd, bd):
    return pl.pallas_call(
        _m5_body,
        out_shape=jax.ShapeDtypeStruct((1, 1), jnp.float32),
    )(mulv, preds, bs, bt, lab, wc1, bc1, wc2, bc2, wd, bd)


def kernel(feats_s, edge_index_s, batch_s, labels_s, feats_t, edge_index_t,
           batch_t, W1_ps, b1_ps, W2_ps, b2_ps, W3_ps, b3_ps,
           W1_pt, b1_pt, W2_pt, b2_pt, W3_pt, b3_pt,
           W1_sh, b1_sh, W2_sh, b2_sh, W3_sh, b3_sh,
           Wc1, bc1, Wc2, bc2, Wd, bd):
    f32 = jnp.float32
    eps = jax.random.normal(jax.random.key(42), (N, H2), f32)
    neg_s = jax.random.randint(jax.random.key(7), (2, E), 0, N)
    neg_t = jax.random.randint(jax.random.key(8), (2, E), 0, N)

    src_s = edge_index_s[0].astype(jnp.int32)
    dst_s = edge_index_s[1].astype(jnp.int32)
    src_t = edge_index_t[0].astype(jnp.int32)
    dst_t = edge_index_t[1].astype(jnp.int32)

    dst_all = jnp.concatenate([dst_s, dst_t])
    src_all = jnp.concatenate([src_s, src_t + N])

    deg = _deg_kernel(dst_all)
    dinv = lax.rsqrt(jnp.clip(deg + 1.0, 1.0, None)).reshape(2, N, 1)

    W1c = jnp.stack([jnp.concatenate([W1_ps, W1_sh], axis=1),
                     jnp.concatenate([W1_pt, W1_sh], axis=1)])
    b1c = jnp.stack([jnp.concatenate([b1_ps, b1_sh]).reshape(1, 2 * H1),
                     jnp.concatenate([b1_pt, b1_sh]).reshape(1, 2 * H1)])
    Z = jnp.zeros((H1, H2), f32)

    def blk(w2a, w3a):
        return jnp.concatenate([w2a, w3a, Z, Z], axis=1)

    def blk2(w2b, w3b):
        return jnp.concatenate([Z, Z, w2b, w3b], axis=1)

    Wblk = jnp.stack([
        jnp.concatenate([blk(W2_ps, W3_ps), blk2(W2_sh, W3_sh)], axis=0),
        jnp.concatenate([blk(W2_pt, W3_pt), blk2(W2_sh, W3_sh)], axis=0)])
    bblk = jnp.stack([
        jnp.concatenate([b2_ps, b3_ps, b2_sh, b3_sh]).reshape(1, 2 * H1),
        jnp.concatenate([b2_pt, b3_pt, b2_sh, b3_sh]).reshape(1, 2 * H1)])

    xs = jnp.stack([feats_s, feats_t])
    zrows = jnp.zeros((16, H1), f32)

    hp1a, hp1b = _m1(xs, W1c, b1c, dinv)
    raw1a, raw1b = _agg_kernel(hp1a.reshape(2 * N, H1),
                               hp1b.reshape(2 * N, H1),
                               src_all, dst_all, zrows)
    hp2a, hp2b = _m2(raw1a, raw1b, hp1a, hp1b, dinv, Wblk, bblk)
    raw2a, raw2b = _agg_kernel(hp2a.reshape(2 * N, H1),
                               hp2b.reshape(2 * N, H1),
                               src_all, dst_all, zrows)
    mulv, zcat = _m3(raw2a, raw2b, hp2a, hp2b, dinv, eps)

    recd = _m4(zcat)

    idx_all = jnp.concatenate([
        src_s * N + dst_s,
        neg_s[0] * N + neg_s[1],
        N * N + src_t * N + dst_t,
        N * N + neg_t[0] * N + neg_t[1]]).astype(jnp.int32)
    preds = _pred_kernel(recd.reshape(2 * N * N), idx_all)

    out = _m5(mulv, preds.reshape(E // 32, H1),
              batch_s.astype(jnp.int32).reshape(N, 1),
              batch_t.astype(jnp.int32).reshape(N, 1),
              labels_s.astype(jnp.int32).reshape(NG, 1),
              Wc1, bc1.reshape(1, 16), Wc2, bc2.reshape(1, NC),
              Wd, bd.reshape(1, 1))
    return out.reshape(())


# speedup vs baseline: 19.9914x; 1.1598x over previous
import functools

import jax
import jax.numpy as jnp
from jax import lax
from jax.experimental import pallas as pl
from jax.experimental.pallas import tpu as pltpu
from jax.experimental.pallas import tpu_sc as plsc

N = 4096
E = 131072
FDIM = 256
H1 = 128
H2 = 64
NG = 64
NC = 10
COEFF_DIFF = 0.1
COEFF_RECON = 0.1
COEFF_DOMAIN = 0.1
NORM = N * N / (2.0 * (N * N - E))

NCORE = 2
NSUB = 16
EPT = E // NSUB
DEG_CHUNK = 256
AGG_CHUNK = 128
PRED_CHUNK = 1024

def _mesh():
    return plsc.VectorSubcoreMesh(core_axis_name="c", subcore_axis_name="s",
                                  num_cores=NCORE, num_subcores=NSUB)


def _make_deg_kernel():
    def body(dst_all, out, acc, idx_i, ones_v, zero_v, sem):
        c = lax.axis_index("c")
        s = lax.axis_index("s")
        for k in range(DEG_CHUNK // 16):
            ones_v[pl.ds(16 * k, 16)] = jnp.ones((16,), jnp.float32)
            zero_v[pl.ds(16 * k, 16)] = jnp.zeros((16,), jnp.float32)
        zrow = s * (N // NSUB)
        pltpu.sync_copy(zero_v, acc.at[pl.ds(zrow, N // NSUB)])
        plsc.subcore_barrier()
        base = c * E + s * EPT

        def step(i, carry):
            pltpu.sync_copy(dst_all.at[pl.ds(base + i * DEG_CHUNK, DEG_CHUNK)],
                            idx_i)
            pltpu.async_copy(ones_v, acc.at[idx_i], sem, add=True).wait()
            return carry

        lax.fori_loop(0, EPT // DEG_CHUNK, step, 0)
        plsc.subcore_barrier()
        pltpu.sync_copy(acc.at[pl.ds(zrow, N // NSUB)],
                        out.at[c, pl.ds(zrow, N // NSUB)])

    return pl.kernel(
        body,
        out_type=jax.ShapeDtypeStruct((NCORE, N), jnp.float32),
        mesh=_mesh(),
        scratch_types=[
            pltpu.VMEM_SHARED((N,), jnp.float32),
            pltpu.VMEM((DEG_CHUNK,), jnp.int32),
            pltpu.VMEM((DEG_CHUNK,), jnp.float32),
            pltpu.VMEM((N // NSUB,), jnp.float32),
            pltpu.SemaphoreType.DMA,
        ],
    )


AGG_NB = 4


def _make_agg_kernel():
    width = H1
    rows_per_tile = N // NSUB
    chunks = EPT // AGG_CHUNK

    def body(hp_a, hp_b, src_all, dst_all, zrows, out_a, out_b,
             acc, i0, i1, i2, i3, j0, j1, j2, j3, b0, b1, b2, b3, zb,
             semi, semg, sema):
        c = lax.axis_index("c")
        s = lax.axis_index("s")
        bufs = (b0, b1, b2, b3)
        idxs = (i0, i1, i2, i3)
        idxd = (j0, j1, j2, j3)
        pltpu.sync_copy(zrows, zb)
        base = c * E + s * EPT
        for phase in range(2):
            hp_h = (hp_a, hp_b)[phase]
            out_h = (out_a, out_b)[phase]
            for k in range(rows_per_tile // 16):
                pltpu.sync_copy(zb, acc.at[pl.ds(s * rows_per_tile + 16 * k,
                                                 16)])
            plsc.subcore_barrier()

            def step(i, carry):
                ls = []
                for b in range(AGG_NB):
                    off = base + (i * AGG_NB + b) * AGG_CHUNK
                    ls.append(pltpu.async_copy(
                        src_all.at[pl.ds(off, AGG_CHUNK)], idxs[b], semi))
                    ls.append(pltpu.async_copy(
                        dst_all.at[pl.ds(off, AGG_CHUNK)], idxd[b], semi))
                gs = []
                for b in range(AGG_NB):
                    ls[2 * b].wait()
                    ls[2 * b + 1].wait()
                    gs.append(pltpu.async_copy(hp_h.at[idxs[b]], bufs[b],
                                               semg))
                ads = []
                for b in range(AGG_NB):
                    gs[b].wait()
                    ads.append(pltpu.async_copy(bufs[b], acc.at[idxd[b]],
                                                sema, add=True))
                for b in range(AGG_NB):
                    ads[b].wait()
                return carry

            lax.fori_loop(0, chunks // AGG_NB, step, 0)
            plsc.subcore_barrier()
            pltpu.sync_copy(acc.at[pl.ds(s * rows_per_tile, rows_per_tile)],
                            out_h.at[c, pl.ds(s * rows_per_tile,
                                              rows_per_tile)])
            plsc.subcore_barrier()

    return pl.kernel(
        body,
        out_type=(jax.ShapeDtypeStruct((NCORE, N, width), jnp.float32),
                  jax.ShapeDtypeStruct((NCORE, N, width), jnp.float32)),
        mesh=_mesh(),
        scratch_types=(
            [pltpu.VMEM_SHARED((N, width), jnp.float32)]
            + [pltpu.VMEM((AGG_CHUNK,), jnp.int32) for _ in range(8)]
            + [pltpu.VMEM((AGG_CHUNK, width), jnp.float32) for _ in range(4)]
            + [pltpu.VMEM((16, width), jnp.float32),
               pltpu.SemaphoreType.DMA,
               pltpu.SemaphoreType.DMA,
               pltpu.SemaphoreType.DMA]
        ),
    )


def _make_pred_kernel():
    ept = (2 * E) // NSUB

    def body(recd_flat, idx_all, out, idx_v, val_v, sem):
        c = lax.axis_index("c")
        s = lax.axis_index("s")
        base = c * (2 * E) + s * ept

        def step(i, carry):
            off = base + i * PRED_CHUNK
            pltpu.sync_copy(idx_all.at[pl.ds(off, PRED_CHUNK)], idx_v)
            pltpu.async_copy(recd_flat.at[idx_v], val_v, sem).wait()
            pltpu.sync_copy(val_v, out.at[pl.ds(off, PRED_CHUNK)])
            return carry

        lax.fori_loop(0, ept // PRED_CHUNK, step, 0)

    return pl.kernel(
        body,
        out_type=jax.ShapeDtypeStruct((4 * E,), jnp.float32),
        mesh=_mesh(),
        scratch_types=[
            pltpu.VMEM((PRED_CHUNK,), jnp.int32),
            pltpu.VMEM((PRED_CHUNK,), jnp.float32),
            pltpu.SemaphoreType.DMA,
        ],
    )


_SC_CACHE = {}


def _deg_kernel(dst_all):
    if "deg" not in _SC_CACHE:
        _SC_CACHE["deg"] = _make_deg_kernel()
    return _SC_CACHE["deg"](dst_all)


def _agg_kernel(hp_a, hp_b, src2d, dst2d, zrows):
    if "agg" not in _SC_CACHE:
        _SC_CACHE["agg"] = _make_agg_kernel()
    return _SC_CACHE["agg"](hp_a, hp_b, src2d, dst2d, zrows)


def _pred_kernel(recd_flat, idx_all):
    if "pred" not in _SC_CACHE:
        _SC_CACHE["pred"] = _make_pred_kernel()
    return _SC_CACHE["pred"](recd_flat, idx_all)


_RB = 512


def _spec_h(gph_i):
    return pl.BlockSpec((1, _RB, H1), lambda gph, i: (gph, i, 0))


def _m1_body(x_ref, w_ref, b_ref, dinv_ref, oa_ref, ob_ref):
    h = jnp.dot(x_ref[0], w_ref[0], preferred_element_type=jnp.float32)
    o = dinv_ref[0] * (h + b_ref[0])
    oa_ref[0] = o[:, :H1]
    ob_ref[0] = o[:, H1:]


def _m1(x, w, b, dinv):
    g = N // _RB
    return pl.pallas_call(
        _m1_body,
        grid=(2, g),
        in_specs=[
            pl.BlockSpec((1, _RB, FDIM), lambda gph, i: (gph, i, 0)),
            pl.BlockSpec((1, FDIM, 2 * H1), lambda gph, i: (gph, 0, 0)),
            pl.BlockSpec((1, 1, 2 * H1), lambda gph, i: (gph, 0, 0)),
            pl.BlockSpec((1, _RB, 1), lambda gph, i: (gph, i, 0)),
        ],
        out_specs=[_spec_h(None), _spec_h(None)],
        out_shape=[jax.ShapeDtypeStruct((2, N, H1), jnp.float32),
                   jax.ShapeDtypeStruct((2, N, H1), jnp.float32)],
    )(x, w, b, dinv)


def _m2_body(ra_ref, rb_ref, ha_ref, hb_ref, dinv_ref, w_ref, b_ref,
             oa_ref, ob_ref):
    dinv = dinv_ref[0]
    h1a = jnp.maximum(dinv * (ra_ref[0] + ha_ref[0]), 0.0)
    h1b = jnp.maximum(dinv * (rb_ref[0] + hb_ref[0]), 0.0)
    w = w_ref[0]
    h = (jnp.dot(h1a, w[:H1], preferred_element_type=jnp.float32)
         + jnp.dot(h1b, w[H1:], preferred_element_type=jnp.float32))
    o = dinv * (h + b_ref[0])
    oa_ref[0] = o[:, :H1]
    ob_ref[0] = o[:, H1:]


def _m2(ra, rb, ha, hb, dinv, w, b):
    g = N // _RB
    return pl.pallas_call(
        _m2_body,
        grid=(2, g),
        in_specs=[
            _spec_h(None), _spec_h(None), _spec_h(None), _spec_h(None),
            pl.BlockSpec((1, _RB, 1), lambda gph, i: (gph, i, 0)),
            pl.BlockSpec((1, 2 * H1, 2 * H1), lambda gph, i: (gph, 0, 0)),
            pl.BlockSpec((1, 1, 2 * H1), lambda gph, i: (gph, 0, 0)),
        ],
        out_specs=[_spec_h(None), _spec_h(None)],
        out_shape=[jax.ShapeDtypeStruct((2, N, H1), jnp.float32),
                   jax.ShapeDtypeStruct((2, N, H1), jnp.float32)],
    )(ra, rb, ha, hb, dinv, w, b)


def _m3_body(ra_ref, rb_ref, ha_ref, hb_ref, dinv_ref, eps_ref,
             mulv_ref, zcat_ref):
    dinv = dinv_ref[0]
    ma = dinv * (ra_ref[0] + ha_ref[0])
    mb = dinv * (rb_ref[0] + hb_ref[0])
    mulv_ref[0] = jnp.concatenate([ma, mb], axis=1)
    eps = eps_ref[...]
    zp = ma[:, :H2] + eps * jnp.exp(ma[:, H2:])
    zh = mb[:, :H2] + eps * jnp.exp(mb[:, H2:])
    zcat_ref[0] = jnp.concatenate([zp, zh], axis=1)


def _m3(ra, rb, ha, hb, dinv, eps):
    g = N // _RB
    return pl.pallas_call(
        _m3_body,
        grid=(2, g),
        in_specs=[
            _spec_h(None), _spec_h(None), _spec_h(None), _spec_h(None),
            pl.BlockSpec((1, _RB, 1), lambda gph, i: (gph, i, 0)),
            pl.BlockSpec((_RB, H2), lambda gph, i: (i, 0)),
        ],
        out_specs=[
            pl.BlockSpec((1, _RB, 2 * H1), lambda gph, i: (gph, i, 0)),
            pl.BlockSpec((1, _RB, H1), lambda gph, i: (gph, i, 0)),
        ],
        out_shape=[
            jax.ShapeDtypeStruct((2, N, 2 * H1), jnp.float32),
            jax.ShapeDtypeStruct((2, N, H1), jnp.float32),
        ],
    )(ra, rb, ha, hb, dinv, eps)


def _m4_body(a_ref, b_ref, o_ref):
    a = a_ref[0]
    for k in range(8):
        o_ref[0, :, k, :] = lax.dot_general(
            a, b_ref[0, k * H1:(k + 1) * H1, :],
            (((1,), (1,)), ((), ())), preferred_element_type=jnp.float32)


def _m4(zcat):
    g = N // _RB
    return pl.pallas_call(
        _m4_body,
        grid=(2, g, 4),
        in_specs=[
            pl.BlockSpec((1, _RB, H1), lambda gph, i, j: (gph, i, 0)),
            pl.BlockSpec((1, 8 * H1, H1), lambda gph, i, j: (gph, j, 0)),
        ],
        out_specs=pl.BlockSpec((1, _RB, 8, H1),
                               lambda gph, i, j: (gph, i, j, 0)),
        out_shape=jax.ShapeDtypeStruct((2, N, N // H1, H1), jnp.float32),
    )(zcat, zcat)


def _m5_body(mulv_ref, preds_ref, bs_ref, bt_ref, lab_ref,
             wc1_ref, bc1_ref, wc2_ref, bc2_ref, wd_ref, bd_ref, o_ref):
    mul_s = mulv_ref[0]
    mul_t = mulv_ref[1]

    preds = preds_ref[...]
    softplus = jnp.log(1.0 + jnp.exp(-jnp.abs(preds)))
    base = jnp.maximum(preds, 0.0) + softplus
    rows = E // H1
    s_pos = jnp.sum(base[:2 * rows])
    s_neg = jnp.sum(base[2 * rows:])
    cost_s = NORM * (s_pos - jnp.sum(preds[:rows])) / (2 * E)
    cost_t = NORM * (s_neg - jnp.sum(preds[2 * rows:3 * rows])) / (2 * E)

    def kld(m, denom):
        mu_p, lv_p = m[:, :H2], m[:, H2:2 * H2]
        mu_h, lv_h = m[:, 2 * H2:3 * H2], m[:, 3 * H2:]
        t = (1.0 + 2.0 * lv_p - mu_p * mu_p - jnp.exp(lv_p) ** 2
             + 1.0 + 2.0 * lv_h - mu_h * mu_h - jnp.exp(lv_h) ** 2)
        return -0.5 / denom * jnp.sum(t) / N

    recon = cost_s + kld(mul_s, N) + cost_t + kld(mul_t, 2 * N)

    sh_s1 = mul_s[:, 2 * H2:3 * H2]
    sh_t1 = mul_t[:, 2 * H2:3 * H2]
    iota_g = lax.broadcasted_iota(jnp.int32, (N, NG), 1)
    oh_s = (bs_ref[...] == iota_g).astype(jnp.float32)
    oh_t = (bt_ref[...] == iota_g).astype(jnp.float32)
    ones_c = jnp.ones((N, 1), jnp.float32)
    cnt_s = jnp.clip(lax.dot_general(oh_s, ones_c, (((0,), (0,)), ((), ())),
                                     preferred_element_type=jnp.float32),
                     1.0, None)
    cnt_t = jnp.clip(lax.dot_general(oh_t, ones_c, (((0,), (0,)), ((), ())),
                                     preferred_element_type=jnp.float32),
                     1.0, None)
    pool_s = lax.dot_general(oh_s, sh_s1, (((0,), (0,)), ((), ())),
                             preferred_element_type=jnp.float32) / cnt_s
    pool_t = lax.dot_general(oh_t, sh_t1, (((0,), (0,)), ((), ())),
                             preferred_element_type=jnp.float32) / cnt_t

    lab1h = (lab_ref[...] == lax.broadcasted_iota(jnp.int32, (NG, NC), 1)
             ).astype(jnp.float32)
    hc = jnp.maximum(
        jnp.dot(pool_s, wc1_ref[...], preferred_element_type=jnp.float32)
        + bc1_ref[...], 0.0)
    logits = (jnp.dot(hc, wc2_ref[...], preferred_element_type=jnp.float32)
              + bc2_ref[...])
    p = 1.0 / (1.0 + jnp.exp(-logits))
    p = jnp.clip(p, 1e-07, 1.0 - 1e-07)
    clf = -jnp.mean(lab1h * jnp.log(p) + (1.0 - lab1h) * jnp.log(1.0 - p))

    def dloss(a, b):
        an = jnp.sqrt(jnp.sum(a * a, axis=1, keepdims=True))
        bn = jnp.sqrt(jnp.sum(b * b, axis=1, keepdims=True))
        a2 = a / (an + 1e-06)
        b2 = b / (bn + 1e-06)
        cmat = lax.dot_general(a2, b2, (((0,), (0,)), ((), ())),
                               preferred_element_type=jnp.float32)
        return jnp.sum(cmat * cmat) / (H2 * H2)

    diff = dloss(mul_s[:, :H2], sh_s1) + dloss(mul_t[:, :H2], sh_t1)

    dp_s = 1.0 / (1.0 + jnp.exp(-(jnp.dot(pool_s, wd_ref[...],
                                          preferred_element_type=jnp.float32)
                                  + bd_ref[...])))
    dp_t = 1.0 / (1.0 + jnp.exp(-(jnp.dot(pool_t, wd_ref[...],
                                          preferred_element_type=jnp.float32)
                                  + bd_ref[...])))
    dp_s = jnp.clip(dp_s, 1e-07, 1.0 - 1e-07)
    dp_t = jnp.clip(dp_t, 1e-07, 1.0 - 1e-07)
    domain = (-jnp.mean(jnp.log(1.0 - dp_s))) + (-jnp.mean(jnp.log(dp_t)))

    total = (clf + COEFF_DIFF * diff + COEFF_RECON * recon
             + COEFF_DOMAIN * domain)
    o_ref[...] = jnp.reshape(total, (1, 1))


def _m5(mulv, preds, bs, bt, lab, wc1, bc1, wc2, bc2, wd, bd):
    return pl.pallas_call(
        _m5_body,
        out_shape=jax.ShapeDtypeStruct((1, 1), jnp.float32),
    )(mulv, preds, bs, bt, lab, wc1, bc1, wc2, bc2, wd, bd)


def kernel(feats_s, edge_index_s, batch_s, labels_s, feats_t, edge_index_t,
           batch_t, W1_ps, b1_ps, W2_ps, b2_ps, W3_ps, b3_ps,
           W1_pt, b1_pt, W2_pt, b2_pt, W3_pt, b3_pt,
           W1_sh, b1_sh, W2_sh, b2_sh, W3_sh, b3_sh,
           Wc1, bc1, Wc2, bc2, Wd, bd):
    f32 = jnp.float32
    eps = jax.random.normal(jax.random.key(42), (N, H2), f32)
    neg_s = jax.random.randint(jax.random.key(7), (2, E), 0, N)
    neg_t = jax.random.randint(jax.random.key(8), (2, E), 0, N)

    src_s = edge_index_s[0].astype(jnp.int32)
    dst_s = edge_index_s[1].astype(jnp.int32)
    src_t = edge_index_t[0].astype(jnp.int32)
    dst_t = edge_index_t[1].astype(jnp.int32)

    dst_all = jnp.concatenate([dst_s, dst_t])
    src_all = jnp.concatenate([src_s, src_t + N])

    deg = _deg_kernel(dst_all)
    dinv = lax.rsqrt(jnp.clip(deg + 1.0, 1.0, None)).reshape(2, N, 1)

    W1c = jnp.stack([jnp.concatenate([W1_ps, W1_sh], axis=1),
                     jnp.concatenate([W1_pt, W1_sh], axis=1)])
    b1c = jnp.stack([jnp.concatenate([b1_ps, b1_sh]).reshape(1, 2 * H1),
                     jnp.concatenate([b1_pt, b1_sh]).reshape(1, 2 * H1)])
    Z = jnp.zeros((H1, H2), f32)

    def blk(w2a, w3a):
        return jnp.concatenate([w2a, w3a, Z, Z], axis=1)

    def blk2(w2b, w3b):
        return jnp.concatenate([Z, Z, w2b, w3b], axis=1)

    Wblk = jnp.stack([
        jnp.concatenate([blk(W2_ps, W3_ps), blk2(W2_sh, W3_sh)], axis=0),
        jnp.concatenate([blk(W2_pt, W3_pt), blk2(W2_sh, W3_sh)], axis=0)])
    bblk = jnp.stack([
        jnp.concatenate([b2_ps, b3_ps, b2_sh, b3_sh]).reshape(1, 2 * H1),
        jnp.concatenate([b2_pt, b3_pt, b2_sh, b3_sh]).reshape(1, 2 * H1)])

    xs = jnp.stack([feats_s, feats_t])
    zrows = jnp.zeros((16, H1), f32)

    hp1a, hp1b = _m1(xs, W1c, b1c, dinv)
    raw1a, raw1b = _agg_kernel(hp1a.reshape(2 * N, H1),
                               hp1b.reshape(2 * N, H1),
                               src_all, dst_all, zrows)
    hp2a, hp2b = _m2(raw1a, raw1b, hp1a, hp1b, dinv, Wblk, bblk)
    raw2a, raw2b = _agg_kernel(hp2a.reshape(2 * N, H1),
                               hp2b.reshape(2 * N, H1),
                               src_all, dst_all, zrows)
    mulv, zcat = _m3(raw2a, raw2b, hp2a, hp2b, dinv, eps)

    recd = _m4(zcat)

    idx_all = jnp.concatenate([
        src_s * N + dst_s,
        neg_s[0] * N + neg_s[1],
        N * N + src_t * N + dst_t,
        N * N + neg_t[0] * N + neg_t[1]]).astype(jnp.int32)
    preds = _pred_kernel(recd.reshape(2 * N * N), idx_all)

    out = _m5(mulv, preds.reshape(E // 32, H1),
              batch_s.astype(jnp.int32).reshape(N, 1),
              batch_t.astype(jnp.int32).reshape(N, 1),
              labels_s.astype(jnp.int32).reshape(NG, 1),
              Wc1, bc1.reshape(1, 16), Wc2, bc2.reshape(1, NC),
              Wd, bd.reshape(1, 1))
    return out.reshape(())


# speedup vs baseline: 21.7557x; 1.0883x over previous
import functools

import jax
import jax.numpy as jnp
from jax import lax
from jax.experimental import pallas as pl
from jax.experimental.pallas import tpu as pltpu
from jax.experimental.pallas import tpu_sc as plsc

N = 4096
E = 131072
FDIM = 256
H1 = 128
H2 = 64
NG = 64
NC = 10
COEFF_DIFF = 0.1
COEFF_RECON = 0.1
COEFF_DOMAIN = 0.1
NORM = N * N / (2.0 * (N * N - E))

NCORE = 2
NSUB = 16
EPT = E // NSUB
DEG_CHUNK = 256
AGG_CHUNK = 128
PRED_CHUNK = 1024

def _mesh():
    return plsc.VectorSubcoreMesh(core_axis_name="c", subcore_axis_name="s",
                                  num_cores=NCORE, num_subcores=NSUB)


def _make_deg_kernel():
    nb = 4

    def body(dst_all, out, acc, i0, i1, i2, i3, ones_v, zero_v, semi, sema):
        c = lax.axis_index("c")
        s = lax.axis_index("s")
        idxv = (i0, i1, i2, i3)
        for k in range(DEG_CHUNK // 16):
            ones_v[pl.ds(16 * k, 16)] = jnp.ones((16,), jnp.float32)
            zero_v[pl.ds(16 * k, 16)] = jnp.zeros((16,), jnp.float32)
        zrow = s * (N // NSUB)
        pltpu.sync_copy(zero_v, acc.at[pl.ds(zrow, N // NSUB)])
        plsc.subcore_barrier()
        base = c * E + s * EPT

        def step(i, carry):
            ls = []
            for b in range(nb):
                off = base + (i * nb + b) * DEG_CHUNK
                ls.append(pltpu.async_copy(
                    dst_all.at[pl.ds(off, DEG_CHUNK)], idxv[b], semi))
            ads = []
            for b in range(nb):
                ls[b].wait()
                ads.append(pltpu.async_copy(ones_v, acc.at[idxv[b]], sema,
                                            add=True))
            for b in range(nb):
                ads[b].wait()
            return carry

        lax.fori_loop(0, EPT // (DEG_CHUNK * nb), step, 0)
        plsc.subcore_barrier()
        pltpu.sync_copy(acc.at[pl.ds(zrow, N // NSUB)],
                        out.at[c, pl.ds(zrow, N // NSUB)])

    return pl.kernel(
        body,
        out_type=jax.ShapeDtypeStruct((NCORE, N), jnp.float32),
        mesh=_mesh(),
        scratch_types=(
            [pltpu.VMEM_SHARED((N,), jnp.float32)]
            + [pltpu.VMEM((DEG_CHUNK,), jnp.int32) for _ in range(4)]
            + [pltpu.VMEM((DEG_CHUNK,), jnp.float32),
               pltpu.VMEM((N // NSUB,), jnp.float32),
               pltpu.SemaphoreType.DMA, pltpu.SemaphoreType.DMA]
        ),
    )


AGG_NB = 4


def _make_agg_kernel():
    width = H1
    rows_per_tile = N // NSUB
    chunks = EPT // AGG_CHUNK

    def body(hp_a, hp_b, src_all, dst_all, zrows, out_a, out_b,
             acc, i0, i1, i2, i3, j0, j1, j2, j3, b0, b1, b2, b3, zb,
             semi, semg, sema):
        c = lax.axis_index("c")
        s = lax.axis_index("s")
        bufs = (b0, b1, b2, b3)
        idxs = (i0, i1, i2, i3)
        idxd = (j0, j1, j2, j3)
        pltpu.sync_copy(zrows, zb)
        base = c * E + s * EPT
        for phase in range(2):
            hp_h = (hp_a, hp_b)[phase]
            out_h = (out_a, out_b)[phase]
            for k in range(rows_per_tile // 16):
                pltpu.sync_copy(zb, acc.at[pl.ds(s * rows_per_tile + 16 * k,
                                                 16)])
            plsc.subcore_barrier()

            def group(i, drain):
                ls = []
                for b in range(AGG_NB):
                    if drain:
                        pltpu.make_async_copy(bufs[b], acc.at[idxd[b]],
                                              sema).wait()
                    off = base + (i * AGG_NB + b) * AGG_CHUNK
                    ls.append(pltpu.async_copy(
                        src_all.at[pl.ds(off, AGG_CHUNK)], idxs[b], semi))
                    ls.append(pltpu.async_copy(
                        dst_all.at[pl.ds(off, AGG_CHUNK)], idxd[b], semi))
                gs = []
                for b in range(AGG_NB):
                    ls[2 * b].wait()
                    ls[2 * b + 1].wait()
                    gs.append(pltpu.async_copy(hp_h.at[idxs[b]], bufs[b],
                                               semg))
                for b in range(AGG_NB):
                    gs[b].wait()
                    pltpu.async_copy(bufs[b], acc.at[idxd[b]], sema,
                                     add=True)

            group(0, False)

            def step(i, carry):
                group(i, True)
                return carry

            lax.fori_loop(1, chunks // AGG_NB, step, 0)
            for b in range(AGG_NB):
                pltpu.make_async_copy(bufs[b], acc.at[idxd[b]], sema).wait()
            plsc.subcore_barrier()
            pltpu.sync_copy(acc.at[pl.ds(s * rows_per_tile, rows_per_tile)],
                            out_h.at[c, pl.ds(s * rows_per_tile,
                                              rows_per_tile)])
            plsc.subcore_barrier()

    return pl.kernel(
        body,
        out_type=(jax.ShapeDtypeStruct((NCORE, N, width), jnp.float32),
                  jax.ShapeDtypeStruct((NCORE, N, width), jnp.float32)),
        mesh=_mesh(),
        scratch_types=(
            [pltpu.VMEM_SHARED((N, width), jnp.float32)]
            + [pltpu.VMEM((AGG_CHUNK,), jnp.int32) for _ in range(8)]
            + [pltpu.VMEM((AGG_CHUNK, width), jnp.float32) for _ in range(4)]
            + [pltpu.VMEM((16, width), jnp.float32),
               pltpu.SemaphoreType.DMA,
               pltpu.SemaphoreType.DMA,
               pltpu.SemaphoreType.DMA]
        ),
    )


def _make_pred_kernel():
    ept = (2 * E) // NSUB

    nb = 4

    def body(recd_flat, idx_all, out, i0, i1, i2, i3, v0, v1, v2, v3,
             semi, semg, semo):
        c = lax.axis_index("c")
        s = lax.axis_index("s")
        idxv = (i0, i1, i2, i3)
        valv = (v0, v1, v2, v3)
        base = c * (2 * E) + s * ept

        def step(i, carry):
            ls = []
            for b in range(nb):
                off = base + (i * nb + b) * PRED_CHUNK
                ls.append(pltpu.async_copy(
                    idx_all.at[pl.ds(off, PRED_CHUNK)], idxv[b], semi))
            gs = []
            for b in range(nb):
                ls[b].wait()
                gs.append(pltpu.async_copy(recd_flat.at[idxv[b]], valv[b],
                                           semg))
            ws = []
            for b in range(nb):
                off = base + (i * nb + b) * PRED_CHUNK
                gs[b].wait()
                ws.append(pltpu.async_copy(valv[b],
                                           out.at[pl.ds(off, PRED_CHUNK)],
                                           semo))
            for b in range(nb):
                ws[b].wait()
            return carry

        lax.fori_loop(0, ept // (PRED_CHUNK * nb), step, 0)

    return pl.kernel(
        body,
        out_type=jax.ShapeDtypeStruct((4 * E,), jnp.float32),
        mesh=_mesh(),
        scratch_types=(
            [pltpu.VMEM((PRED_CHUNK,), jnp.int32) for _ in range(4)]
            + [pltpu.VMEM((PRED_CHUNK,), jnp.float32) for _ in range(4)]
            + [pltpu.SemaphoreType.DMA, pltpu.SemaphoreType.DMA,
               pltpu.SemaphoreType.DMA]
        ),
    )


_SC_CACHE = {}


def _deg_kernel(dst_all):
    if "deg" not in _SC_CACHE:
        _SC_CACHE["deg"] = _make_deg_kernel()
    return _SC_CACHE["deg"](dst_all)


def _agg_kernel(hp_a, hp_b, src2d, dst2d, zrows):
    if "agg" not in _SC_CACHE:
        _SC_CACHE["agg"] = _make_agg_kernel()
    return _SC_CACHE["agg"](hp_a, hp_b, src2d, dst2d, zrows)


def _pred_kernel(recd_flat, idx_all):
    if "pred" not in _SC_CACHE:
        _SC_CACHE["pred"] = _make_pred_kernel()
    return _SC_CACHE["pred"](recd_flat, idx_all)


_RB = 512


def _spec_h(gph_i):
    return pl.BlockSpec((1, _RB, H1), lambda gph, i: (gph, i, 0))


def _m1_body(x_ref, w_ref, b_ref, dinv_ref, oa_ref, ob_ref):
    h = jnp.dot(x_ref[0], w_ref[0], preferred_element_type=jnp.float32)
    o = dinv_ref[0] * (h + b_ref[0])
    oa_ref[0] = o[:, :H1]
    ob_ref[0] = o[:, H1:]


def _m1(x, w, b, dinv):
    g = N // _RB
    return pl.pallas_call(
        _m1_body,
        grid=(2, g),
        in_specs=[
            pl.BlockSpec((1, _RB, FDIM), lambda gph, i: (gph, i, 0)),
            pl.BlockSpec((1, FDIM, 2 * H1), lambda gph, i: (gph, 0, 0)),
            pl.BlockSpec((1, 1, 2 * H1), lambda gph, i: (gph, 0, 0)),
            pl.BlockSpec((1, _RB, 1), lambda gph, i: (gph, i, 0)),
        ],
        out_specs=[_spec_h(None), _spec_h(None)],
        out_shape=[jax.ShapeDtypeStruct((2, N, H1), jnp.float32),
                   jax.ShapeDtypeStruct((2, N, H1), jnp.float32)],
    )(x, w, b, dinv)


def _m2_body(ra_ref, rb_ref, ha_ref, hb_ref, dinv_ref, w_ref, b_ref,
             oa_ref, ob_ref):
    dinv = dinv_ref[0]
    h1a = jnp.maximum(dinv * (ra_ref[0] + ha_ref[0]), 0.0)
    h1b = jnp.maximum(dinv * (rb_ref[0] + hb_ref[0]), 0.0)
    w = w_ref[0]
    h = (jnp.dot(h1a, w[:H1], preferred_element_type=jnp.float32)
         + jnp.dot(h1b, w[H1:], preferred_element_type=jnp.float32))
    o = dinv * (h + b_ref[0])
    oa_ref[0] = o[:, :H1]
    ob_ref[0] = o[:, H1:]


def _m2(ra, rb, ha, hb, dinv, w, b):
    g = N // _RB
    return pl.pallas_call(
        _m2_body,
        grid=(2, g),
        in_specs=[
            _spec_h(None), _spec_h(None), _spec_h(None), _spec_h(None),
            pl.BlockSpec((1, _RB, 1), lambda gph, i: (gph, i, 0)),
            pl.BlockSpec((1, 2 * H1, 2 * H1), lambda gph, i: (gph, 0, 0)),
            pl.BlockSpec((1, 1, 2 * H1), lambda gph, i: (gph, 0, 0)),
        ],
        out_specs=[_spec_h(None), _spec_h(None)],
        out_shape=[jax.ShapeDtypeStruct((2, N, H1), jnp.float32),
                   jax.ShapeDtypeStruct((2, N, H1), jnp.float32)],
    )(ra, rb, ha, hb, dinv, w, b)


def _m3_body(ra_ref, rb_ref, ha_ref, hb_ref, dinv_ref, eps_ref,
             mulv_ref, zcat_ref):
    dinv = dinv_ref[0]
    ma = dinv * (ra_ref[0] + ha_ref[0])
    mb = dinv * (rb_ref[0] + hb_ref[0])
    mulv_ref[0] = jnp.concatenate([ma, mb], axis=1)
    eps = eps_ref[...]
    zp = ma[:, :H2] + eps * jnp.exp(ma[:, H2:])
    zh = mb[:, :H2] + eps * jnp.exp(mb[:, H2:])
    zcat_ref[0] = jnp.concatenate([zp, zh], axis=1)


def _m3(ra, rb, ha, hb, dinv, eps):
    g = N // _RB
    return pl.pallas_call(
        _m3_body,
        grid=(2, g),
        in_specs=[
            _spec_h(None), _spec_h(None), _spec_h(None), _spec_h(None),
            pl.BlockSpec((1, _RB, 1), lambda gph, i: (gph, i, 0)),
            pl.BlockSpec((_RB, H2), lambda gph, i: (i, 0)),
        ],
        out_specs=[
            pl.BlockSpec((1, _RB, 2 * H1), lambda gph, i: (gph, i, 0)),
            pl.BlockSpec((1, _RB, H1), lambda gph, i: (gph, i, 0)),
        ],
        out_shape=[
            jax.ShapeDtypeStruct((2, N, 2 * H1), jnp.float32),
            jax.ShapeDtypeStruct((2, N, H1), jnp.float32),
        ],
    )(ra, rb, ha, hb, dinv, eps)


def _m4_body(a_ref, b_ref, o_ref):
    a = a_ref[0]
    for k in range(8):
        o_ref[0, :, k, :] = lax.dot_general(
            a, b_ref[0, k * H1:(k + 1) * H1, :],
            (((1,), (1,)), ((), ())), preferred_element_type=jnp.float32)


def _m4(zcat):
    g = N // _RB
    return pl.pallas_call(
        _m4_body,
        grid=(2, g, 4),
        in_specs=[
            pl.BlockSpec((1, _RB, H1), lambda gph, i, j: (gph, i, 0)),
            pl.BlockSpec((1, 8 * H1, H1), lambda gph, i, j: (gph, j, 0)),
        ],
        out_specs=pl.BlockSpec((1, _RB, 8, H1),
                               lambda gph, i, j: (gph, i, j, 0)),
        out_shape=jax.ShapeDtypeStruct((2, N, N // H1, H1), jnp.float32),
    )(zcat, zcat)


def _m5_body(mulv_ref, preds_ref, bs_ref, bt_ref, lab_ref,
             wc1_ref, bc1_ref, wc2_ref, bc2_ref, wd_ref, bd_ref, o_ref):
    mul_s = mulv_ref[0]
    mul_t = mulv_ref[1]

    preds = preds_ref[...]
    softplus = jnp.log(1.0 + jnp.exp(-jnp.abs(preds)))
    base = jnp.maximum(preds, 0.0) + softplus
    rows = E // H1
    s_pos = jnp.sum(base[:2 * rows])
    s_neg = jnp.sum(base[2 * rows:])
    cost_s = NORM * (s_pos - jnp.sum(preds[:rows])) / (2 * E)
    cost_t = NORM * (s_neg - jnp.sum(preds[2 * rows:3 * rows])) / (2 * E)

    def kld(m, denom):
        mu_p, lv_p = m[:, :H2], m[:, H2:2 * H2]
        mu_h, lv_h = m[:, 2 * H2:3 * H2], m[:, 3 * H2:]
        t = (1.0 + 2.0 * lv_p - mu_p * mu_p - jnp.exp(lv_p) ** 2
             + 1.0 + 2.0 * lv_h - mu_h * mu_h - jnp.exp(lv_h) ** 2)
        return -0.5 / denom * jnp.sum(t) / N

    recon = cost_s + kld(mul_s, N) + cost_t + kld(mul_t, 2 * N)

    sh_s1 = mul_s[:, 2 * H2:3 * H2]
    sh_t1 = mul_t[:, 2 * H2:3 * H2]
    iota_g = lax.broadcasted_iota(jnp.int32, (N, NG), 1)
    oh_s = (bs_ref[...] == iota_g).astype(jnp.float32)
    oh_t = (bt_ref[...] == iota_g).astype(jnp.float32)
    ones_c = jnp.ones((N, 1), jnp.float32)
    cnt_s = jnp.clip(lax.dot_general(oh_s, ones_c, (((0,), (0,)), ((), ())),
                                     preferred_element_type=jnp.float32),
                     1.0, None)
    cnt_t = jnp.clip(lax.dot_general(oh_t, ones_c, (((0,), (0,)), ((), ())),
                                     preferred_element_type=jnp.float32),
                     1.0, None)
    pool_s = lax.dot_general(oh_s, sh_s1, (((0,), (0,)), ((), ())),
                             preferred_element_type=jnp.float32) / cnt_s
    pool_t = lax.dot_general(oh_t, sh_t1, (((0,), (0,)), ((), ())),
                             preferred_element_type=jnp.float32) / cnt_t

    lab1h = (lab_ref[...] == lax.broadcasted_iota(jnp.int32, (NG, NC), 1)
             ).astype(jnp.float32)
    hc = jnp.maximum(
        jnp.dot(pool_s, wc1_ref[...], preferred_element_type=jnp.float32)
        + bc1_ref[...], 0.0)
    logits = (jnp.dot(hc, wc2_ref[...], preferred_element_type=jnp.float32)
              + bc2_ref[...])
    p = 1.0 / (1.0 + jnp.exp(-logits))
    p = jnp.clip(p, 1e-07, 1.0 - 1e-07)
    clf = -jnp.mean(lab1h * jnp.log(p) + (1.0 - lab1h) * jnp.log(1.0 - p))

    def dloss(a, b):
        an = jnp.sqrt(jnp.sum(a * a, axis=1, keepdims=True))
        bn = jnp.sqrt(jnp.sum(b * b, axis=1, keepdims=True))
        a2 = a / (an + 1e-06)
        b2 = b / (bn + 1e-06)
        cmat = lax.dot_general(a2, b2, (((0,), (0,)), ((), ())),
                               preferred_element_type=jnp.float32)
        return jnp.sum(cmat * cmat) / (H2 * H2)

    diff = dloss(mul_s[:, :H2], sh_s1) + dloss(mul_t[:, :H2], sh_t1)

    dp_s = 1.0 / (1.0 + jnp.exp(-(jnp.dot(pool_s, wd_ref[...],
                                          preferred_element_type=jnp.float32)
                                  + bd_ref[...])))
    dp_t = 1.0 / (1.0 + jnp.exp(-(jnp.dot(pool_t, wd_ref[...],
                                          preferred_element_type=jnp.float32)
                                  + bd_ref[...])))
    dp_s = jnp.clip(dp_s, 1e-07, 1.0 - 1e-07)
    dp_t = jnp.clip(dp_t, 1e-07, 1.0 - 1e-07)
    domain = (-jnp.mean(jnp.log(1.0 - dp_s))) + (-jnp.mean(jnp.log(dp_t)))

    total = (clf + COEFF_DIFF * diff + COEFF_RECON * recon
             + COEFF_DOMAIN * domain)
    o_ref[...] = jnp.reshape(total, (1, 1))


def _m5(mulv, preds, bs, bt, lab, wc1, bc1, wc2, bc2, wd, bd):
    return pl.pallas_call(
        _m5_body,
        out_shape=jax.ShapeDtypeStruct((1, 1), jnp.float32),
    )(mulv, preds, bs, bt, lab, wc1, bc1, wc2, bc2, wd, bd)


def kernel(feats_s, edge_index_s, batch_s, labels_s, feats_t, edge_index_t,
           batch_t, W1_ps, b1_ps, W2_ps, b2_ps, W3_ps, b3_ps,
           W1_pt, b1_pt, W2_pt, b2_pt, W3_pt, b3_pt,
           W1_sh, b1_sh, W2_sh, b2_sh, W3_sh, b3_sh,
           Wc1, bc1, Wc2, bc2, Wd, bd):
    f32 = jnp.float32
    eps = jax.random.normal(jax.random.key(42), (N, H2), f32)
    neg_s = jax.random.randint(jax.random.key(7), (2, E), 0, N)
    neg_t = jax.random.randint(jax.random.key(8), (2, E), 0, N)

    src_s = edge_index_s[0].astype(jnp.int32)
    dst_s = edge_index_s[1].astype(jnp.int32)
    src_t = edge_index_t[0].astype(jnp.int32)
    dst_t = edge_index_t[1].astype(jnp.int32)

    dst_all = jnp.concatenate([dst_s, dst_t])
    src_all = jnp.concatenate([src_s, src_t + N])

    deg = _deg_kernel(dst_all)
    dinv = lax.rsqrt(jnp.clip(deg + 1.0, 1.0, None)).reshape(2, N, 1)

    W1c = jnp.stack([jnp.concatenate([W1_ps, W1_sh], axis=1),
                     jnp.concatenate([W1_pt, W1_sh], axis=1)])
    b1c = jnp.stack([jnp.concatenate([b1_ps, b1_sh]).reshape(1, 2 * H1),
                     jnp.concatenate([b1_pt, b1_sh]).reshape(1, 2 * H1)])
    Z = jnp.zeros((H1, H2), f32)

    def blk(w2a, w3a):
        return jnp.concatenate([w2a, w3a, Z, Z], axis=1)

    def blk2(w2b, w3b):
        return jnp.concatenate([Z, Z, w2b, w3b], axis=1)

    Wblk = jnp.stack([
        jnp.concatenate([blk(W2_ps, W3_ps), blk2(W2_sh, W3_sh)], axis=0),
        jnp.concatenate([blk(W2_pt, W3_pt), blk2(W2_sh, W3_sh)], axis=0)])
    bblk = jnp.stack([
        jnp.concatenate([b2_ps, b3_ps, b2_sh, b3_sh]).reshape(1, 2 * H1),
        jnp.concatenate([b2_pt, b3_pt, b2_sh, b3_sh]).reshape(1, 2 * H1)])

    xs = jnp.stack([feats_s, feats_t])
    zrows = jnp.zeros((16, H1), f32)

    hp1a, hp1b = _m1(xs, W1c, b1c, dinv)
    raw1a, raw1b = _agg_kernel(hp1a.reshape(2 * N, H1),
                               hp1b.reshape(2 * N, H1),
                               src_all, dst_all, zrows)
    hp2a, hp2b = _m2(raw1a, raw1b, hp1a, hp1b, dinv, Wblk, bblk)
    raw2a, raw2b = _agg_kernel(hp2a.reshape(2 * N, H1),
                               hp2b.reshape(2 * N, H1),
                               src_all, dst_all, zrows)
    mulv, zcat = _m3(raw2a, raw2b, hp2a, hp2b, dinv, eps)

    recd = _m4(zcat)

    idx_all = jnp.concatenate([
        src_s * N + dst_s,
        neg_s[0] * N + neg_s[1],
        N * N + src_t * N + dst_t,
        N * N + neg_t[0] * N + neg_t[1]]).astype(jnp.int32)
    preds = _pred_kernel(recd.reshape(2 * N * N), idx_all)

    out = _m5(mulv, preds.reshape(E // 32, H1),
              batch_s.astype(jnp.int32).reshape(N, 1),
              batch_t.astype(jnp.int32).reshape(N, 1),
              labels_s.astype(jnp.int32).reshape(NG, 1),
              Wc1, bc1.reshape(1, 16), Wc2, bc2.reshape(1, NC),
              Wd, bd.reshape(1, 1))
    return out.reshape(())


# speedup vs baseline: 23.2211x; 1.0674x over previous
import jax
import jax.numpy as jnp
from jax import lax
from jax.experimental import pallas as pl
from jax.experimental.pallas import tpu as pltpu
from jax.experimental.pallas import tpu_sc as plsc

N = 4096
E = 131072
FDIM = 256
H1 = 128
H2 = 64
NG = 64
NC = 10
COEFF_DIFF = 0.1
COEFF_RECON = 0.1
COEFF_DOMAIN = 0.1
NORM = N * N / (2.0 * (N * N - E))

NCORE = 2
NSUB = 16
NW = NCORE * NSUB
DEG_CHUNK = 256
AGG_CHUNK = 128
PRED_CHUNK = 1024
AGG_NB = 4


def _mesh():
    return plsc.VectorSubcoreMesh(core_axis_name="c", subcore_axis_name="s",
                                  num_cores=NCORE, num_subcores=NSUB)


def _make_deg_kernel():
    nb = 4
    ept = E // NSUB

    def body(dst_all, out, acc, i0, i1, i2, i3, ones_v, zero_v, semi, sema):
        c = lax.axis_index("c")
        s = lax.axis_index("s")
        idxv = (i0, i1, i2, i3)
        for k in range(DEG_CHUNK // 16):
            ones_v[pl.ds(16 * k, 16)] = jnp.ones((16,), jnp.float32)
            zero_v[pl.ds(16 * k, 16)] = jnp.zeros((16,), jnp.float32)
        zrow = s * (N // NSUB)
        pltpu.sync_copy(zero_v, acc.at[pl.ds(zrow, N // NSUB)])
        plsc.subcore_barrier()
        base = c * E + s * ept

        def step(i, carry):
            ls = []
            for b in range(nb):
                off = base + (i * nb + b) * DEG_CHUNK
                ls.append(pltpu.async_copy(
                    dst_all.at[pl.ds(off, DEG_CHUNK)], idxv[b], semi))
            ads = []
            for b in range(nb):
                ls[b].wait()
                ads.append(pltpu.async_copy(ones_v, acc.at[idxv[b]], sema,
                                            add=True))
            for b in range(nb):
                ads[b].wait()
            return carry

        lax.fori_loop(0, ept // (DEG_CHUNK * nb), step, 0)
        plsc.subcore_barrier()
        pltpu.sync_copy(acc.at[pl.ds(zrow, N // NSUB)],
                        out.at[c, pl.ds(zrow, N // NSUB)])

    return pl.kernel(
        body,
        out_type=jax.ShapeDtypeStruct((NCORE, N), jnp.float32),
        mesh=_mesh(),
        scratch_types=(
            [pltpu.VMEM_SHARED((N,), jnp.float32)]
            + [pltpu.VMEM((DEG_CHUNK,), jnp.int32) for _ in range(4)]
            + [pltpu.VMEM((DEG_CHUNK,), jnp.float32),
               pltpu.VMEM((N // NSUB,), jnp.float32),
               pltpu.SemaphoreType.DMA, pltpu.SemaphoreType.DMA]
        ),
    )


def _make_agg_kernel():
    width = H1
    rows_per_tile = N // NSUB
    ept = E // NW
    chunks = ept // AGG_CHUNK

    def body(hp_a, hp_b, src_g, dst_g, zrows, out_a, out_b,
             acc, i0, i1, i2, i3, j0, j1, j2, j3, b0, b1, b2, b3, zb,
             semi, semg, sema):
        c = lax.axis_index("c")
        s = lax.axis_index("s")
        bufs = (b0, b1, b2, b3)
        idxs = (i0, i1, i2, i3)
        idxd = (j0, j1, j2, j3)
        pltpu.sync_copy(zrows, zb)
        base = (s * NCORE + c) * ept
        for phase in range(2):
            hp_h = (hp_a, hp_b)[phase]
            out_h = (out_a, out_b)[phase]
            for k in range(rows_per_tile // 16):
                pltpu.sync_copy(zb, acc.at[pl.ds(s * rows_per_tile + 16 * k,
                                                 16)])
            plsc.subcore_barrier()

            def group(i, drain):
                ls = []
                for b in range(AGG_NB):
                    if drain:
                        pltpu.make_async_copy(bufs[b], acc.at[idxd[b]],
                                              sema).wait()
                    off = base + (i * AGG_NB + b) * AGG_CHUNK
                    ls.append(pltpu.async_copy(
                        src_g.at[pl.ds(off, AGG_CHUNK)], idxs[b], semi))
                    ls.append(pltpu.async_copy(
                        dst_g.at[pl.ds(off, AGG_CHUNK)], idxd[b], semi))
                gs = []
                for b in range(AGG_NB):
                    ls[2 * b].wait()
                    ls[2 * b + 1].wait()
                    gs.append(pltpu.async_copy(hp_h.at[idxs[b]], bufs[b],
                                               semg))
                for b in range(AGG_NB):
                    gs[b].wait()
                    pltpu.async_copy(bufs[b], acc.at[idxd[b]], sema,
                                     add=True)

            group(0, False)

            def step(i, carry):
                group(i, True)
                return carry

            lax.fori_loop(1, chunks // AGG_NB, step, 0)
            for b in range(AGG_NB):
                pltpu.make_async_copy(bufs[b], acc.at[idxd[b]], sema).wait()
            plsc.subcore_barrier()
            pltpu.sync_copy(acc.at[pl.ds(s * rows_per_tile, rows_per_tile)],
                            out_h.at[c, pl.ds(s * rows_per_tile,
                                              rows_per_tile)])
            plsc.subcore_barrier()

    return pl.kernel(
        body,
        out_type=(jax.ShapeDtypeStruct((NCORE, N, width), jnp.float32),
                  jax.ShapeDtypeStruct((NCORE, N, width), jnp.float32)),
        mesh=_mesh(),
        scratch_types=(
            [pltpu.VMEM_SHARED((N, width), jnp.float32)]
            + [pltpu.VMEM((AGG_CHUNK,), jnp.int32) for _ in range(8)]
            + [pltpu.VMEM((AGG_CHUNK, width), jnp.float32) for _ in range(4)]
            + [pltpu.VMEM((16, width), jnp.float32),
               pltpu.SemaphoreType.DMA,
               pltpu.SemaphoreType.DMA,
               pltpu.SemaphoreType.DMA]
        ),
    )


def _make_pred_kernel():
    nb = 4
    ept = (2 * E) // NW

    def body(recd_flat, idx_g, out, i0, i1, i2, i3, v0, v1, v2, v3,
             semi, semg, semo):
        c = lax.axis_index("c")
        s = lax.axis_index("s")
        idxv = (i0, i1, i2, i3)
        valv = (v0, v1, v2, v3)
        base = (s * NCORE + c) * ept

        def step(i, carry):
            ls = []
            for b in range(nb):
                off = base + (i * nb + b) * PRED_CHUNK
                ls.append(pltpu.async_copy(
                    idx_g.at[pl.ds(off, PRED_CHUNK)], idxv[b], semi))
            gs = []
            for b in range(nb):
                ls[b].wait()
                gs.append(pltpu.async_copy(recd_flat.at[idxv[b]], valv[b],
                                           semg))
            ws = []
            for b in range(nb):
                off = base + (i * nb + b) * PRED_CHUNK
                gs[b].wait()
                ws.append(pltpu.async_copy(valv[b],
                                           out.at[pl.ds(off, PRED_CHUNK)],
                                           semo))
            for b in range(nb):
                ws[b].wait()
            return carry

        lax.fori_loop(0, ept // (PRED_CHUNK * nb), step, 0)

    return pl.kernel(
        body,
        out_type=jax.ShapeDtypeStruct((2 * E,), jnp.float32),
        mesh=_mesh(),
        scratch_types=(
            [pltpu.VMEM((PRED_CHUNK,), jnp.int32) for _ in range(4)]
            + [pltpu.VMEM((PRED_CHUNK,), jnp.float32) for _ in range(4)]
            + [pltpu.SemaphoreType.DMA, pltpu.SemaphoreType.DMA,
               pltpu.SemaphoreType.DMA]
        ),
    )


_SC_CACHE = {}


def _deg_kernel(dst_all):
    if "deg" not in _SC_CACHE:
        _SC_CACHE["deg"] = _make_deg_kernel()
    return _SC_CACHE["deg"](dst_all)


def _agg_kernel(hp_a, hp_b, src_g, dst_g, zrows):
    if "agg" not in _SC_CACHE:
        _SC_CACHE["agg"] = _make_agg_kernel()
    return _SC_CACHE["agg"](hp_a, hp_b, src_g, dst_g, zrows)


def _pred_kernel(recd_flat, idx_g):
    if "pred" not in _SC_CACHE:
        _SC_CACHE["pred"] = _make_pred_kernel()
    return _SC_CACHE["pred"](recd_flat, idx_g)


_RB = 512


def _spec_h():
    return pl.BlockSpec((_RB, H1), lambda i: (i, 0))


def _spec_r():
    return pl.BlockSpec((NCORE, _RB, H1), lambda i: (0, i, 0))


def _m1_body(x_ref, w_ref, b_ref, dinv_ref, oa_ref, ob_ref):
    h = jnp.dot(x_ref[...], w_ref[...], preferred_element_type=jnp.float32)
    o = dinv_ref[...] * (h + b_ref[...])
    oa_ref[...] = o[:, :H1]
    ob_ref[...] = o[:, H1:]


def _m1(x, w, b, dinv):
    g = N // _RB
    return pl.pallas_call(
        _m1_body,
        grid=(g,),
        in_specs=[
            pl.BlockSpec((_RB, FDIM), lambda i: (i, 0)),
            pl.BlockSpec((FDIM, 2 * H1), lambda i: (0, 0)),
            pl.BlockSpec((1, 2 * H1), lambda i: (0, 0)),
            pl.BlockSpec((_RB, 1), lambda i: (i, 0)),
        ],
        out_specs=[_spec_h(), _spec_h()],
        out_shape=[jax.ShapeDtypeStruct((N, H1), jnp.float32),
                   jax.ShapeDtypeStruct((N, H1), jnp.float32)],
    )(x, w, b, dinv)


def _m2_body(ra_ref, rb_ref, ha_ref, hb_ref, dinv_ref, w_ref, b_ref,
             oa_ref, ob_ref):
    dinv = dinv_ref[...]
    h1a = jnp.maximum(dinv * (ra_ref[0] + ra_ref[1] + ha_ref[...]), 0.0)
    h1b = jnp.maximum(dinv * (rb_ref[0] + rb_ref[1] + hb_ref[...]), 0.0)
    w = w_ref[...]
    h = (jnp.dot(h1a, w[:H1], preferred_element_type=jnp.float32)
         + jnp.dot(h1b, w[H1:], preferred_element_type=jnp.float32))
    o = dinv * (h + b_ref[...])
    oa_ref[...] = o[:, :H1]
    ob_ref[...] = o[:, H1:]


def _m2(ra, rb, ha, hb, dinv, w, b):
    g = N // _RB
    return pl.pallas_call(
        _m2_body,
        grid=(g,),
        in_specs=[
            _spec_r(), _spec_r(), _spec_h(), _spec_h(),
            pl.BlockSpec((_RB, 1), lambda i: (i, 0)),
            pl.BlockSpec((2 * H1, 2 * H1), lambda i: (0, 0)),
            pl.BlockSpec((1, 2 * H1), lambda i: (0, 0)),
        ],
        out_specs=[_spec_h(), _spec_h()],
        out_shape=[jax.ShapeDtypeStruct((N, H1), jnp.float32),
                   jax.ShapeDtypeStruct((N, H1), jnp.float32)],
    )(ra, rb, ha, hb, dinv, w, b)


def _m3_body(ra_ref, rb_ref, ha_ref, hb_ref, dinv_ref, eps_ref,
             mulv_ref, zcat_ref):
    dinv = dinv_ref[...]
    ma = dinv * (ra_ref[0] + ra_ref[1] + ha_ref[...])
    mb = dinv * (rb_ref[0] + rb_ref[1] + hb_ref[...])
    mulv_ref[...] = jnp.concatenate([ma, mb], axis=1)
    eps = eps_ref[...]
    zp = ma[:, :H2] + eps * jnp.exp(ma[:, H2:])
    zh = mb[:, :H2] + eps * jnp.exp(mb[:, H2:])
    zcat_ref[...] = jnp.concatenate([zp, zh], axis=1)


def _m3(ra, rb, ha, hb, dinv, eps):
    g = N // _RB
    return pl.pallas_call(
        _m3_body,
        grid=(g,),
        in_specs=[
            _spec_r(), _spec_r(), _spec_h(), _spec_h(),
            pl.BlockSpec((_RB, 1), lambda i: (i, 0)),
            pl.BlockSpec((_RB, H2), lambda i: (i, 0)),
        ],
        out_specs=[
            pl.BlockSpec((_RB, 2 * H1), lambda i: (i, 0)),
            pl.BlockSpec((_RB, H1), lambda i: (i, 0)),
        ],
        out_shape=[
            jax.ShapeDtypeStruct((N, 2 * H1), jnp.float32),
            jax.ShapeDtypeStruct((N, H1), jnp.float32),
        ],
    )(ra, rb, ha, hb, dinv, eps)


def _m4_body(a_ref, b_ref, o_ref):
    a = a_ref[...]
    for k in range(8):
        o_ref[:, k, :] = lax.dot_general(
            a, b_ref[k * H1:(k + 1) * H1, :],
            (((1,), (1,)), ((), ())), preferred_element_type=jnp.float32)


def _m4(zcat):
    g = N // _RB
    return pl.pallas_call(
        _m4_body,
        grid=(g, 4),
        in_specs=[
            pl.BlockSpec((_RB, H1), lambda i, j: (i, 0)),
            pl.BlockSpec((8 * H1, H1), lambda i, j: (j, 0)),
        ],
        out_specs=pl.BlockSpec((_RB, 8, H1), lambda i, j: (i, j, 0)),
        out_shape=jax.ShapeDtypeStruct((N, N // H1, H1), jnp.float32),
    )(zcat, zcat)


def _m5_body(muls_ref, mult_ref, ps_ref, pt_ref, bs_ref, bt_ref, lab_ref,
             wc1_ref, bc1_ref, wc2_ref, bc2_ref, wd_ref, bd_ref, o_ref):
    mul_s = muls_ref[...]
    mul_t = mult_ref[...]

    rows = E // H1

    def bce_cost(p_ref):
        preds = p_ref[...]
        softplus = jnp.log(1.0 + jnp.exp(-jnp.abs(preds)))
        base = jnp.maximum(preds, 0.0) + softplus
        return NORM * (jnp.sum(base) - jnp.sum(preds[:rows])) / (2 * E)

    def kld(m, denom):
        mu_p, lv_p = m[:, :H2], m[:, H2:2 * H2]
        mu_h, lv_h = m[:, 2 * H2:3 * H2], m[:, 3 * H2:]
        t = (1.0 + 2.0 * lv_p - mu_p * mu_p - jnp.exp(lv_p) ** 2
             + 1.0 + 2.0 * lv_h - mu_h * mu_h - jnp.exp(lv_h) ** 2)
        return -0.5 / denom * jnp.sum(t) / N

    recon = (bce_cost(ps_ref) + kld(mul_s, N)
             + bce_cost(pt_ref) + kld(mul_t, 2 * N))

    sh_s1 = mul_s[:, 2 * H2:3 * H2]
    sh_t1 = mul_t[:, 2 * H2:3 * H2]
    iota_g = lax.broadcasted_iota(jnp.int32, (N, NG), 1)
    oh_s = (bs_ref[...] == iota_g).astype(jnp.float32)
    oh_t = (bt_ref[...] == iota_g).astype(jnp.float32)
    ones_c = jnp.ones((N, 1), jnp.float32)
    cnt_s = jnp.clip(lax.dot_general(oh_s, ones_c, (((0,), (0,)), ((), ())),
                                     preferred_element_type=jnp.float32),
                     1.0, None)
    cnt_t = jnp.clip(lax.dot_general(oh_t, ones_c, (((0,), (0,)), ((), ())),
                                     preferred_element_type=jnp.float32),
                     1.0, None)
    pool_s = lax.dot_general(oh_s, sh_s1, (((0,), (0,)), ((), ())),
                             preferred_element_type=jnp.float32) / cnt_s
    pool_t = lax.dot_general(oh_t, sh_t1, (((0,), (0,)), ((), ())),
                             preferred_element_type=jnp.float32) / cnt_t

    lab1h = (lab_ref[...] == lax.broadcasted_iota(jnp.int32, (NG, NC), 1)
             ).astype(jnp.float32)
    hc = jnp.maximum(
        jnp.dot(pool_s, wc1_ref[...], preferred_element_type=jnp.float32)
        + bc1_ref[...], 0.0)
    logits = (jnp.dot(hc, wc2_ref[...], preferred_element_type=jnp.float32)
              + bc2_ref[...])
    p = 1.0 / (1.0 + jnp.exp(-logits))
    p = jnp.clip(p, 1e-07, 1.0 - 1e-07)
    clf = -jnp.mean(lab1h * jnp.log(p) + (1.0 - lab1h) * jnp.log(1.0 - p))

    def dloss(a, b):
        an = jnp.sqrt(jnp.sum(a * a, axis=1, keepdims=True))
        bn = jnp.sqrt(jnp.sum(b * b, axis=1, keepdims=True))
        a2 = a / (an + 1e-06)
        b2 = b / (bn + 1e-06)
        cmat = lax.dot_general(a2, b2, (((0,), (0,)), ((), ())),
                               preferred_element_type=jnp.float32)
        return jnp.sum(cmat * cmat) / (H2 * H2)

    diff = dloss(mul_s[:, :H2], sh_s1) + dloss(mul_t[:, :H2], sh_t1)

    dp_s = 1.0 / (1.0 + jnp.exp(-(jnp.dot(pool_s, wd_ref[...],
                                          preferred_element_type=jnp.float32)
                                  + bd_ref[...])))
    dp_t = 1.0 / (1.0 + jnp.exp(-(jnp.dot(pool_t, wd_ref[...],
                                          preferred_element_type=jnp.float32)
                                  + bd_ref[...])))
    dp_s = jnp.clip(dp_s, 1e-07, 1.0 - 1e-07)
    dp_t = jnp.clip(dp_t, 1e-07, 1.0 - 1e-07)
    domain = (-jnp.mean(jnp.log(1.0 - dp_s))) + (-jnp.mean(jnp.log(dp_t)))

    total = (clf + COEFF_DIFF * diff + COEFF_RECON * recon
             + COEFF_DOMAIN * domain)
    o_ref[...] = jnp.reshape(total, (1, 1))


def _m5(mul_s, mul_t, ps, pt, bs, bt, lab, wc1, bc1, wc2, bc2, wd, bd):
    return pl.pallas_call(
        _m5_body,
        out_shape=jax.ShapeDtypeStruct((1, 1), jnp.float32),
    )(mul_s, mul_t, ps, pt, bs, bt, lab, wc1, bc1, wc2, bc2, wd, bd)


def kernel(feats_s, edge_index_s, batch_s, labels_s, feats_t, edge_index_t,
           batch_t, W1_ps, b1_ps, W2_ps, b2_ps, W3_ps, b3_ps,
           W1_pt, b1_pt, W2_pt, b2_pt, W3_pt, b3_pt,
           W1_sh, b1_sh, W2_sh, b2_sh, W3_sh, b3_sh,
           Wc1, bc1, Wc2, bc2, Wd, bd):
    f32 = jnp.float32
    eps = jax.random.normal(jax.random.key(42), (N, H2), f32)
    neg_s = jax.random.randint(jax.random.key(7), (2, E), 0, N)
    neg_t = jax.random.randint(jax.random.key(8), (2, E), 0, N)

    src_s = edge_index_s[0].astype(jnp.int32)
    dst_s = edge_index_s[1].astype(jnp.int32)
    src_t = edge_index_t[0].astype(jnp.int32)
    dst_t = edge_index_t[1].astype(jnp.int32)

    deg = _deg_kernel(jnp.concatenate([dst_s, dst_t]))
    dinv2 = lax.rsqrt(jnp.clip(deg + 1.0, 1.0, None)).reshape(2, N, 1)

    Z = jnp.zeros((H1, H2), f32)

    def wblk(w2a, w3a, w2b, w3b):
        return jnp.concatenate([
            jnp.concatenate([w2a, w3a, Z, Z], axis=1),
            jnp.concatenate([Z, Z, w2b, w3b], axis=1)], axis=0)

    zrows = jnp.zeros((16, H1), f32)
    graphs = []
    for gph, (x, src, dst, w1a, b1a, w2, b2, w3, b3) in enumerate((
            (feats_s, src_s, dst_s, W1_ps, b1_ps, W2_ps, b2_ps, W3_ps, b3_ps),
            (feats_t, src_t, dst_t, W1_pt, b1_pt, W2_pt, b2_pt, W3_pt, b3_pt),
    )):
        dinv = dinv2[gph]
        w1c = jnp.concatenate([w1a, W1_sh], axis=1)
        b1c = jnp.concatenate([b1a, b1_sh]).reshape(1, 2 * H1)
        wb = wblk(w2, w3, W2_sh, W3_sh)
        bb = jnp.concatenate([b2, b3, b2_sh, b3_sh]).reshape(1, 2 * H1)

        h1a, h1b = _m1(x, w1c, b1c, dinv)
        r1a, r1b = _agg_kernel(h1a, h1b, src, dst, zrows)
        h2a, h2b = _m2(r1a, r1b, h1a, h1b, dinv, wb, bb)
        r2a, r2b = _agg_kernel(h2a, h2b, src, dst, zrows)
        mulv, zcat = _m3(r2a, r2b, h2a, h2b, dinv, eps)
        recd = _m4(zcat)
        neg = (neg_s, neg_t)[gph]
        idx_g = jnp.concatenate([src * N + dst,
                                 neg[0] * N + neg[1]]).astype(jnp.int32)
        preds = _pred_kernel(recd.reshape(N * N), idx_g)
        graphs.append((mulv, preds))

    out = _m5(graphs[0][0], graphs[1][0],
              graphs[0][1].reshape(2 * E // H1, H1),
              graphs[1][1].reshape(2 * E // H1, H1),
              batch_s.astype(jnp.int32).reshape(N, 1),
              batch_t.astype(jnp.int32).reshape(N, 1),
              labels_s.astype(jnp.int32).reshape(NG, 1),
              Wc1, bc1.reshape(1, 16), Wc2, bc2.reshape(1, NC),
              Wd, bd.reshape(1, 1))
    return out.reshape(())


# speedup vs baseline: 23.4252x; 1.0088x over previous
import jax
import jax.numpy as jnp
from jax import lax
from jax.experimental import pallas as pl
from jax.experimental.pallas import tpu as pltpu
from jax.experimental.pallas import tpu_sc as plsc

N = 4096
E = 131072
FDIM = 256
H1 = 128
H2 = 64
NG = 64
NC = 10
COEFF_DIFF = 0.1
COEFF_RECON = 0.1
COEFF_DOMAIN = 0.1
NORM = N * N / (2.0 * (N * N - E))

NCORE = 2
NSUB = 16
NW = NCORE * NSUB
DEG_CHUNK = 256
AGG_CHUNK = 128
PRED_CHUNK = 1024
AGG_NB = 4


def _mesh():
    return plsc.VectorSubcoreMesh(core_axis_name="c", subcore_axis_name="s",
                                  num_cores=NCORE, num_subcores=NSUB)


def _make_deg_kernel():
    nb = 4
    ept = E // NSUB

    def body(dst_all, out, acc, i0, i1, i2, i3, ones_v, zero_v, semi, sema):
        c = lax.axis_index("c")
        s = lax.axis_index("s")
        idxv = (i0, i1, i2, i3)
        for k in range(DEG_CHUNK // 16):
            ones_v[pl.ds(16 * k, 16)] = jnp.ones((16,), jnp.float32)
            zero_v[pl.ds(16 * k, 16)] = jnp.zeros((16,), jnp.float32)
        zrow = s * (N // NSUB)
        pltpu.sync_copy(zero_v, acc.at[pl.ds(zrow, N // NSUB)])
        plsc.subcore_barrier()
        base = c * E + s * ept

        def step(i, carry):
            ls = []
            for b in range(nb):
                off = base + (i * nb + b) * DEG_CHUNK
                ls.append(pltpu.async_copy(
                    dst_all.at[pl.ds(off, DEG_CHUNK)], idxv[b], semi))
            ads = []
            for b in range(nb):
                ls[b].wait()
                ads.append(pltpu.async_copy(ones_v, acc.at[idxv[b]], sema,
                                            add=True))
            for b in range(nb):
                ads[b].wait()
            return carry

        lax.fori_loop(0, ept // (DEG_CHUNK * nb), step, 0)
        plsc.subcore_barrier()
        pltpu.sync_copy(acc.at[pl.ds(zrow, N // NSUB)],
                        out.at[c, pl.ds(zrow, N // NSUB)])

    return pl.kernel(
        body,
        out_type=jax.ShapeDtypeStruct((NCORE, N), jnp.float32),
        mesh=_mesh(),
        scratch_types=(
            [pltpu.VMEM_SHARED((N,), jnp.float32)]
            + [pltpu.VMEM((DEG_CHUNK,), jnp.int32) for _ in range(4)]
            + [pltpu.VMEM((DEG_CHUNK,), jnp.float32),
               pltpu.VMEM((N // NSUB,), jnp.float32),
               pltpu.SemaphoreType.DMA, pltpu.SemaphoreType.DMA]
        ),
    )


def _make_agg_kernel():
    width = H1
    rows_per_tile = N // NSUB
    ept = E // NW
    chunks = ept // AGG_CHUNK

    def body(hp_a, hp_b, src_g, dst_g, zrows, out_a, out_b,
             acc, i0, i1, i2, i3, j0, j1, j2, j3, b0, b1, b2, b3, zb,
             semi, semg, sema):
        c = lax.axis_index("c")
        s = lax.axis_index("s")
        bufs = (b0, b1, b2, b3)
        idxs = (i0, i1, i2, i3)
        idxd = (j0, j1, j2, j3)
        pltpu.sync_copy(zrows, zb)
        base = (s * NCORE + c) * ept
        for phase in range(2):
            hp_h = (hp_a, hp_b)[phase]
            out_h = (out_a, out_b)[phase]
            zs = [pltpu.async_copy(
                zb, acc.at[pl.ds(s * rows_per_tile + 16 * k, 16)], semg)
                for k in range(rows_per_tile // 16)]
            for z in zs:
                z.wait()
            plsc.subcore_barrier()

            def group(i, drain):
                ls = []
                for b in range(AGG_NB):
                    if drain:
                        pltpu.make_async_copy(bufs[b], acc.at[idxd[b]],
                                              sema).wait()
                    off = base + (i * AGG_NB + b) * AGG_CHUNK
                    ls.append(pltpu.async_copy(
                        src_g.at[pl.ds(off, AGG_CHUNK)], idxs[b], semi))
                    ls.append(pltpu.async_copy(
                        dst_g.at[pl.ds(off, AGG_CHUNK)], idxd[b], semi))
                gs = []
                for b in range(AGG_NB):
                    ls[2 * b].wait()
                    ls[2 * b + 1].wait()
                    gs.append(pltpu.async_copy(hp_h.at[idxs[b]], bufs[b],
                                               semg))
                for b in range(AGG_NB):
                    gs[b].wait()
                    pltpu.async_copy(bufs[b], acc.at[idxd[b]], sema,
                                     add=True)

            group(0, False)

            def step(i, carry):
                group(i, True)
                return carry

            lax.fori_loop(1, chunks // AGG_NB, step, 0)
            for b in range(AGG_NB):
                pltpu.make_async_copy(bufs[b], acc.at[idxd[b]], sema).wait()
            plsc.subcore_barrier()
            pltpu.sync_copy(acc.at[pl.ds(s * rows_per_tile, rows_per_tile)],
                            out_h.at[c, pl.ds(s * rows_per_tile,
                                              rows_per_tile)])
            plsc.subcore_barrier()

    return pl.kernel(
        body,
        out_type=(jax.ShapeDtypeStruct((NCORE, N, width), jnp.float32),
                  jax.ShapeDtypeStruct((NCORE, N, width), jnp.float32)),
        mesh=_mesh(),
        scratch_types=(
            [pltpu.VMEM_SHARED((N, width), jnp.float32)]
            + [pltpu.VMEM((AGG_CHUNK,), jnp.int32) for _ in range(8)]
            + [pltpu.VMEM((AGG_CHUNK, width), jnp.float32) for _ in range(4)]
            + [pltpu.VMEM((16, width), jnp.float32),
               pltpu.SemaphoreType.DMA,
               pltpu.SemaphoreType.DMA,
               pltpu.SemaphoreType.DMA]
        ),
    )


def _make_pred_kernel():
    nb = 4
    ept = (2 * E) // NW

    def body(recd_flat, idx_g, out, i0, i1, i2, i3, v0, v1, v2, v3,
             semi, semg, semo):
        c = lax.axis_index("c")
        s = lax.axis_index("s")
        idxv = (i0, i1, i2, i3)
        valv = (v0, v1, v2, v3)
        base = (s * NCORE + c) * ept

        def step(i, carry):
            ls = []
            for b in range(nb):
                off = base + (i * nb + b) * PRED_CHUNK
                ls.append(pltpu.async_copy(
                    idx_g.at[pl.ds(off, PRED_CHUNK)], idxv[b], semi))
            gs = []
            for b in range(nb):
                ls[b].wait()
                gs.append(pltpu.async_copy(recd_flat.at[idxv[b]], valv[b],
                                           semg))
            ws = []
            for b in range(nb):
                off = base + (i * nb + b) * PRED_CHUNK
                gs[b].wait()
                ws.append(pltpu.async_copy(valv[b],
                                           out.at[pl.ds(off, PRED_CHUNK)],
                                           semo))
            for b in range(nb):
                ws[b].wait()
            return carry

        lax.fori_loop(0, ept // (PRED_CHUNK * nb), step, 0)

    return pl.kernel(
        body,
        out_type=jax.ShapeDtypeStruct((2 * E,), jnp.float32),
        mesh=_mesh(),
        scratch_types=(
            [pltpu.VMEM((PRED_CHUNK,), jnp.int32) for _ in range(4)]
            + [pltpu.VMEM((PRED_CHUNK,), jnp.float32) for _ in range(4)]
            + [pltpu.SemaphoreType.DMA, pltpu.SemaphoreType.DMA,
               pltpu.SemaphoreType.DMA]
        ),
    )


_SC_CACHE = {}


def _deg_kernel(dst_all):
    if "deg" not in _SC_CACHE:
        _SC_CACHE["deg"] = _make_deg_kernel()
    return _SC_CACHE["deg"](dst_all)


def _agg_kernel(hp_a, hp_b, src_g, dst_g, zrows):
    if "agg" not in _SC_CACHE:
        _SC_CACHE["agg"] = _make_agg_kernel()
    return _SC_CACHE["agg"](hp_a, hp_b, src_g, dst_g, zrows)


def _pred_kernel(recd_flat, idx_g):
    if "pred" not in _SC_CACHE:
        _SC_CACHE["pred"] = _make_pred_kernel()
    return _SC_CACHE["pred"](recd_flat, idx_g)


_RB = 512


def _spec_h():
    return pl.BlockSpec((_RB, H1), lambda i: (i, 0))


def _spec_r():
    return pl.BlockSpec((NCORE, _RB, H1), lambda i: (0, i, 0))


def _m1_body(x_ref, w_ref, b_ref, dinv_ref, oa_ref, ob_ref):
    h = jnp.dot(x_ref[...], w_ref[...], preferred_element_type=jnp.float32)
    o = dinv_ref[...] * (h + b_ref[...])
    oa_ref[...] = o[:, :H1]
    ob_ref[...] = o[:, H1:]


def _m1(x, w, b, dinv):
    g = N // _RB
    return pl.pallas_call(
        _m1_body,
        grid=(g,),
        in_specs=[
            pl.BlockSpec((_RB, FDIM), lambda i: (i, 0)),
            pl.BlockSpec((FDIM, 2 * H1), lambda i: (0, 0)),
            pl.BlockSpec((1, 2 * H1), lambda i: (0, 0)),
            pl.BlockSpec((_RB, 1), lambda i: (i, 0)),
        ],
        out_specs=[_spec_h(), _spec_h()],
        out_shape=[jax.ShapeDtypeStruct((N, H1), jnp.float32),
                   jax.ShapeDtypeStruct((N, H1), jnp.float32)],
    )(x, w, b, dinv)


def _m2_body(ra_ref, rb_ref, ha_ref, hb_ref, dinv_ref, w_ref, b_ref,
             oa_ref, ob_ref):
    dinv = dinv_ref[...]
    h1a = jnp.maximum(dinv * (ra_ref[0] + ra_ref[1] + ha_ref[...]), 0.0)
    h1b = jnp.maximum(dinv * (rb_ref[0] + rb_ref[1] + hb_ref[...]), 0.0)
    w = w_ref[...]
    h = (jnp.dot(h1a, w[:H1], preferred_element_type=jnp.float32)
         + jnp.dot(h1b, w[H1:], preferred_element_type=jnp.float32))
    o = dinv * (h + b_ref[...])
    oa_ref[...] = o[:, :H1]
    ob_ref[...] = o[:, H1:]


def _m2(ra, rb, ha, hb, dinv, w, b):
    g = N // _RB
    return pl.pallas_call(
        _m2_body,
        grid=(g,),
        in_specs=[
            _spec_r(), _spec_r(), _spec_h(), _spec_h(),
            pl.BlockSpec((_RB, 1), lambda i: (i, 0)),
            pl.BlockSpec((2 * H1, 2 * H1), lambda i: (0, 0)),
            pl.BlockSpec((1, 2 * H1), lambda i: (0, 0)),
        ],
        out_specs=[_spec_h(), _spec_h()],
        out_shape=[jax.ShapeDtypeStruct((N, H1), jnp.float32),
                   jax.ShapeDtypeStruct((N, H1), jnp.float32)],
    )(ra, rb, ha, hb, dinv, w, b)


def _m3_body(ra_ref, rb_ref, ha_ref, hb_ref, dinv_ref, eps_ref,
             mulv_ref, zcat_ref):
    dinv = dinv_ref[...]
    ma = dinv * (ra_ref[0] + ra_ref[1] + ha_ref[...])
    mb = dinv * (rb_ref[0] + rb_ref[1] + hb_ref[...])
    mulv_ref[...] = jnp.concatenate([ma, mb], axis=1)
    eps = eps_ref[...]
    zp = ma[:, :H2] + eps * jnp.exp(ma[:, H2:])
    zh = mb[:, :H2] + eps * jnp.exp(mb[:, H2:])
    zcat_ref[...] = jnp.concatenate([zp, zh], axis=1)


def _m3(ra, rb, ha, hb, dinv, eps):
    g = N // _RB
    return pl.pallas_call(
        _m3_body,
        grid=(g,),
        in_specs=[
            _spec_r(), _spec_r(), _spec_h(), _spec_h(),
            pl.BlockSpec((_RB, 1), lambda i: (i, 0)),
            pl.BlockSpec((_RB, H2), lambda i: (i, 0)),
        ],
        out_specs=[
            pl.BlockSpec((_RB, 2 * H1), lambda i: (i, 0)),
            pl.BlockSpec((_RB, H1), lambda i: (i, 0)),
        ],
        out_shape=[
            jax.ShapeDtypeStruct((N, 2 * H1), jnp.float32),
            jax.ShapeDtypeStruct((N, H1), jnp.float32),
        ],
    )(ra, rb, ha, hb, dinv, eps)


def _m4_body(a_ref, b_ref, o_ref):
    a = a_ref[...]
    for k in range(8):
        o_ref[:, k, :] = lax.dot_general(
            a, b_ref[k * H1:(k + 1) * H1, :],
            (((1,), (1,)), ((), ())), preferred_element_type=jnp.float32)


def _m4(zcat):
    g = N // _RB
    return pl.pallas_call(
        _m4_body,
        grid=(g, 4),
        in_specs=[
            pl.BlockSpec((_RB, H1), lambda i, j: (i, 0)),
            pl.BlockSpec((8 * H1, H1), lambda i, j: (j, 0)),
        ],
        out_specs=pl.BlockSpec((_RB, 8, H1), lambda i, j: (i, j, 0)),
        out_shape=jax.ShapeDtypeStruct((N, N // H1, H1), jnp.float32),
    )(zcat, zcat)


def _m5_body(muls_ref, mult_ref, ps_ref, pt_ref, bs_ref, bt_ref, lab_ref,
             wc1_ref, bc1_ref, wc2_ref, bc2_ref, wd_ref, bd_ref, o_ref):
    mul_s = muls_ref[...]
    mul_t = mult_ref[...]

    rows = E // H1

    def bce_cost(p_ref):
        preds = p_ref[...]
        softplus = jnp.log(1.0 + jnp.exp(-jnp.abs(preds)))
        base = jnp.maximum(preds, 0.0) + softplus
        return NORM * (jnp.sum(base) - jnp.sum(preds[:rows])) / (2 * E)

    def kld(m, denom):
        mu_p, lv_p = m[:, :H2], m[:, H2:2 * H2]
        mu_h, lv_h = m[:, 2 * H2:3 * H2], m[:, 3 * H2:]
        t = (1.0 + 2.0 * lv_p - mu_p * mu_p - jnp.exp(lv_p) ** 2
             + 1.0 + 2.0 * lv_h - mu_h * mu_h - jnp.exp(lv_h) ** 2)
        return -0.5 / denom * jnp.sum(t) / N

    recon = (bce_cost(ps_ref) + kld(mul_s, N)
             + bce_cost(pt_ref) + kld(mul_t, 2 * N))

    sh_s1 = mul_s[:, 2 * H2:3 * H2]
    sh_t1 = mul_t[:, 2 * H2:3 * H2]
    iota_g = lax.broadcasted_iota(jnp.int32, (N, NG), 1)
    oh_s = (bs_ref[...] == iota_g).astype(jnp.float32)
    oh_t = (bt_ref[...] == iota_g).astype(jnp.float32)
    ones_c = jnp.ones((N, 1), jnp.float32)
    cnt_s = jnp.clip(lax.dot_general(oh_s, ones_c, (((0,), (0,)), ((), ())),
                                     preferred_element_type=jnp.float32),
                     1.0, None)
    cnt_t = jnp.clip(lax.dot_general(oh_t, ones_c, (((0,), (0,)), ((), ())),
                                     preferred_element_type=jnp.float32),
                     1.0, None)
    pool_s = lax.dot_general(oh_s, sh_s1, (((0,), (0,)), ((), ())),
                             preferred_element_type=jnp.float32) / cnt_s
    pool_t = lax.dot_general(oh_t, sh_t1, (((0,), (0,)), ((), ())),
                             preferred_element_type=jnp.float32) / cnt_t

    lab1h = (lab_ref[...] == lax.broadcasted_iota(jnp.int32, (NG, NC), 1)
             ).astype(jnp.float32)
    hc = jnp.maximum(
        jnp.dot(pool_s, wc1_ref[...], preferred_element_type=jnp.float32)
        + bc1_ref[...], 0.0)
    logits = (jnp.dot(hc, wc2_ref[...], preferred_element_type=jnp.float32)
              + bc2_ref[...])
    p = 1.0 / (1.0 + jnp.exp(-logits))
    p = jnp.clip(p, 1e-07, 1.0 - 1e-07)
    clf = -jnp.mean(lab1h * jnp.log(p) + (1.0 - lab1h) * jnp.log(1.0 - p))

    def dloss(a, b):
        an = jnp.sqrt(jnp.sum(a * a, axis=1, keepdims=True))
        bn = jnp.sqrt(jnp.sum(b * b, axis=1, keepdims=True))
        a2 = a / (an + 1e-06)
        b2 = b / (bn + 1e-06)
        cmat = lax.dot_general(a2, b2, (((0,), (0,)), ((), ())),
                               preferred_element_type=jnp.float32)
        return jnp.sum(cmat * cmat) / (H2 * H2)

    diff = dloss(mul_s[:, :H2], sh_s1) + dloss(mul_t[:, :H2], sh_t1)

    dp_s = 1.0 / (1.0 + jnp.exp(-(jnp.dot(pool_s, wd_ref[...],
                                          preferred_element_type=jnp.float32)
                                  + bd_ref[...])))
    dp_t = 1.0 / (1.0 + jnp.exp(-(jnp.dot(pool_t, wd_ref[...],
                                          preferred_element_type=jnp.float32)
                                  + bd_ref[...])))
    dp_s = jnp.clip(dp_s, 1e-07, 1.0 - 1e-07)
    dp_t = jnp.clip(dp_t, 1e-07, 1.0 - 1e-07)
    domain = (-jnp.mean(jnp.log(1.0 - dp_s))) + (-jnp.mean(jnp.log(dp_t)))

    total = (clf + COEFF_DIFF * diff + COEFF_RECON * recon
             + COEFF_DOMAIN * domain)
    o_ref[...] = jnp.reshape(total, (1, 1))


def _m5(mul_s, mul_t, ps, pt, bs, bt, lab, wc1, bc1, wc2, bc2, wd, bd):
    return pl.pallas_call(
        _m5_body,
        out_shape=jax.ShapeDtypeStruct((1, 1), jnp.float32),
    )(mul_s, mul_t, ps, pt, bs, bt, lab, wc1, bc1, wc2, bc2, wd, bd)


def kernel(feats_s, edge_index_s, batch_s, labels_s, feats_t, edge_index_t,
           batch_t, W1_ps, b1_ps, W2_ps, b2_ps, W3_ps, b3_ps,
           W1_pt, b1_pt, W2_pt, b2_pt, W3_pt, b3_pt,
           W1_sh, b1_sh, W2_sh, b2_sh, W3_sh, b3_sh,
           Wc1, bc1, Wc2, bc2, Wd, bd):
    f32 = jnp.float32
    eps = jax.random.normal(jax.random.key(42), (N, H2), f32)
    neg_s = jax.random.randint(jax.random.key(7), (2, E), 0, N)
    neg_t = jax.random.randint(jax.random.key(8), (2, E), 0, N)

    src_s = edge_index_s[0].astype(jnp.int32)
    dst_s = edge_index_s[1].astype(jnp.int32)
    src_t = edge_index_t[0].astype(jnp.int32)
    dst_t = edge_index_t[1].astype(jnp.int32)

    deg = _deg_kernel(jnp.concatenate([dst_s, dst_t]))
    dinv2 = lax.rsqrt(jnp.clip(deg + 1.0, 1.0, None)).reshape(2, N, 1)

    Z = jnp.zeros((H1, H2), f32)

    def wblk(w2a, w3a, w2b, w3b):
        return jnp.concatenate([
            jnp.concatenate([w2a, w3a, Z, Z], axis=1),
            jnp.concatenate([Z, Z, w2b, w3b], axis=1)], axis=0)

    zrows = jnp.zeros((16, H1), f32)
    graphs = []
    for gph, (x, src, dst, w1a, b1a, w2, b2, w3, b3) in enumerate((
            (feats_s, src_s, dst_s, W1_ps, b1_ps, W2_ps, b2_ps, W3_ps, b3_ps),
            (feats_t, src_t, dst_t, W1_pt, b1_pt, W2_pt, b2_pt, W3_pt, b3_pt),
    )):
        dinv = dinv2[gph]
        w1c = jnp.concatenate([w1a, W1_sh], axis=1)
        b1c = jnp.concatenate([b1a, b1_sh]).reshape(1, 2 * H1)
        wb = wblk(w2, w3, W2_sh, W3_sh)
        bb = jnp.concatenate([b2, b3, b2_sh, b3_sh]).reshape(1, 2 * H1)

        h1a, h1b = _m1(x, w1c, b1c, dinv)
        r1a, r1b = _agg_kernel(h1a, h1b, src, dst, zrows)
        h2a, h2b = _m2(r1a, r1b, h1a, h1b, dinv, wb, bb)
        r2a, r2b = _agg_kernel(h2a, h2b, src, dst, zrows)
        mulv, zcat = _m3(r2a, r2b, h2a, h2b, dinv, eps)
        recd = _m4(zcat)
        neg = (neg_s, neg_t)[gph]
        idx_g = jnp.concatenate([src * N + dst,
                                 neg[0] * N + neg[1]]).astype(jnp.int32)
        preds = _pred_kernel(recd.reshape(N * N), idx_g)
        graphs.append((mulv, preds))

    out = _m5(graphs[0][0], graphs[1][0],
              graphs[0][1].reshape(2 * E // H1, H1),
              graphs[1][1].reshape(2 * E // H1, H1),
              batch_s.astype(jnp.int32).reshape(N, 1),
              batch_t.astype(jnp.int32).reshape(N, 1),
              labels_s.astype(jnp.int32).reshape(NG, 1),
              Wc1, bc1.reshape(1, 16), Wc2, bc2.reshape(1, NC),
              Wd, bd.reshape(1, 1))
    return out.reshape(())
